# baseline (speedup 1.0000x reference)
import functools

import jax
import jax.numpy as jnp
from jax import lax
from jax.experimental import pallas as pl
from jax.experimental.pallas import tpu as pltpu
from jax.experimental.pallas import tpu_sc as plsc

N = 10000
E = 320000
IN_CH = 128
H = 200
HP = 256
EDGE_DIM = 4
G = 64
N_PAD = 10240
NW = 32
NPT = N_PAD // NW
CAP = 16384
KB = 64
CH = 2000

@functools.cache
def _mesh():
    return plsc.VectorSubcoreMesh(core_axis_name="c", subcore_axis_name="s",
                                  num_cores=2, num_subcores=16)
_sc_params = pltpu.CompilerParams(needs_layout_passes=False)


def _sc_lazy(**kw):
    def deco(body):
        @functools.cache
        def make():
            return pl.kernel(body, mesh=_mesh(), compiler_params=_sc_params, **kw)
        return lambda *args: make()(*args)
    return deco

def _leaky(v, slope=0.01):
    return jnp.where(v >= 0, v, slope * v)


def _elu(v):
    return jnp.where(v > 0, v, jnp.exp(jnp.minimum(v, 0.0)) - 1.0)

def _wid():
    return lax.axis_index("c") * 16 + lax.axis_index("s")


def _embed_body(x_ref, w1_ref, b1_ref, w1xp_ref, w2p_ref, attr_ref, attl_ref,
                w1e_ref, xv_ref, u_ref, w_ref, ar_ref, bmax_ref):
    i = pl.program_id(0)
    xv = _leaky(jnp.dot(x_ref[...], w1_ref[...].T, preferred_element_type=jnp.float32)
                + b1_ref[...])
    xv_ref[...] = xv
    u = jnp.dot(xv, w1xp_ref[...].T, preferred_element_type=jnp.float32)
    u_ref[...] = u
    w_ref[...] = jnp.dot(xv, w2p_ref[...].T, preferred_element_type=jnp.float32)
    ar_ref[...] = jnp.sum(xv * attr_ref[...][None, :], axis=1)[:, None]
    w1e = w1e_ref[...]
    vmin = jnp.sum(jnp.minimum(w1e, 0.0), axis=1)
    vmax = jnp.sum(jnp.maximum(w1e, 0.0), axis=1)
    attl = attl_ref[...]
    u200 = u[:, :H]
    tb = jnp.maximum(attl[None, :] * _leaky(u200 + vmin[None, :]),
                     attl[None, :] * _leaky(u200 + vmax[None, :]))
    bmx = jnp.max(jnp.sum(tb, axis=1))

    @pl.when(i == 0)
    def _():
        bmax_ref[...] = jnp.full((8, 128), -1e30, jnp.float32)
    bmax_ref[...] = jnp.maximum(bmax_ref[...], bmx)


def _embed(x, w1, b1, w1xp, w2p, attr, attl, w1e):
    blk = 1000
    return pl.pallas_call(
        _embed_body,
        grid=(N // blk,),
        in_specs=[
            pl.BlockSpec((blk, IN_CH), lambda i: (i, 0)),
            pl.BlockSpec((H, IN_CH), lambda i: (0, 0)),
            pl.BlockSpec((H,), lambda i: (0,)),
            pl.BlockSpec((HP, H), lambda i: (0, 0)),
            pl.BlockSpec((HP, H), lambda i: (0, 0)),
            pl.BlockSpec((H,), lambda i: (0,)),
            pl.BlockSpec((H,), lambda i: (0,)),
            pl.BlockSpec((H, EDGE_DIM), lambda i: (0, 0)),
        ],
        out_specs=[
            pl.BlockSpec((blk, H), lambda i: (i, 0)),
            pl.BlockSpec((blk, HP), lambda i: (i, 0)),
            pl.BlockSpec((blk, HP), lambda i: (i, 0)),
            pl.BlockSpec((blk, 1), lambda i: (i, 0)),
            pl.BlockSpec((8, 128), lambda i: (0, 0)),
        ],
        out_shape=[
            jax.ShapeDtypeStruct((N, H), jnp.float32),
            jax.ShapeDtypeStruct((N, HP), jnp.float32),
            jax.ShapeDtypeStruct((N, HP), jnp.float32),
            jax.ShapeDtypeStruct((N, 1), jnp.float32),
            jax.ShapeDtypeStruct((8, 128), jnp.float32),
        ],
    )(x, w1, b1, w1xp, w2p, attr, attl, w1e)


def _proj_body(xv_ref, lwp_ref, asrc_ref, adst_ref, h_ref, as_ref, ad_ref, asmax_ref):
    i = pl.program_id(0)
    h = jnp.dot(xv_ref[...], lwp_ref[...].T, preferred_element_type=jnp.float32)
    h_ref[...] = h
    h200 = h[:, :H]
    a_s = jnp.sum(h200 * asrc_ref[...][None, :], axis=1)
    as_ref[...] = a_s[:, None]
    ad_ref[...] = jnp.sum(h200 * adst_ref[...][None, :], axis=1)[:, None]

    @pl.when(i == 0)
    def _():
        asmax_ref[...] = jnp.full((8, 128), -1e30, jnp.float32)
    asmax_ref[...] = jnp.maximum(asmax_ref[...], jnp.max(a_s))


def _proj(xv, lwp, asrc, adst):
    blk = 1000
    return pl.pallas_call(
        _proj_body,
        grid=(N // blk,),
        in_specs=[
            pl.BlockSpec((blk, H), lambda i: (i, 0)),
            pl.BlockSpec((HP, H), lambda i: (0, 0)),
            pl.BlockSpec((H,), lambda i: (0,)),
            pl.BlockSpec((H,), lambda i: (0,)),
        ],
        out_specs=[
            pl.BlockSpec((blk, HP), lambda i: (i, 0)),
            pl.BlockSpec((blk, 1), lambda i: (i, 0)),
            pl.BlockSpec((blk, 1), lambda i: (i, 0)),
            pl.BlockSpec((8, 128), lambda i: (0, 0)),
        ],
        out_shape=[
            jax.ShapeDtypeStruct((N, HP), jnp.float32),
            jax.ShapeDtypeStruct((N, 1), jnp.float32),
            jax.ShapeDtypeStruct((N, 1), jnp.float32),
            jax.ShapeDtypeStruct((8, 128), jnp.float32),
        ],
    )(xv, lwp, asrc, adst)


def _gru_body(msg_ref, bias_ref, xv_ref, wr_ref, wz_ref, wn_ref,
              vr_ref, vz_ref, vn_ref, br_ref, bz_ref, bn_ref,
              cr_ref, cz_ref, cn_ref, out_ref):
    h = _elu(msg_ref[...][:, :H] + bias_ref[...][None, :])
    xv = xv_ref[...]
    dot = lambda a, b: jnp.dot(a, b.T, preferred_element_type=jnp.float32)
    i_r = dot(h, wr_ref[...]) + br_ref[...]
    i_z = dot(h, wz_ref[...]) + bz_ref[...]
    i_n = dot(h, wn_ref[...]) + bn_ref[...]
    h_r = dot(xv, vr_ref[...]) + cr_ref[...]
    h_z = dot(xv, vz_ref[...]) + cz_ref[...]
    h_n = dot(xv, vn_ref[...]) + cn_ref[...]
    r = jax.nn.sigmoid(i_r + h_r)
    z = jax.nn.sigmoid(i_z + h_z)
    n = jnp.tanh(i_n + r * h_n)
    out_ref[...] = jax.nn.relu((1.0 - z) * n + z * xv)


def _gru(msg, bias, xv, gru_p):
    blk = 1000
    wih, whh, bih, bhh = gru_p["wih"], gru_p["whh"], gru_p["bih"], gru_p["bhh"]
    args = [msg, bias, xv,
            wih[:H], wih[H:2 * H], wih[2 * H:],
            whh[:H], whh[H:2 * H], whh[2 * H:],
            bih[:H], bih[H:2 * H], bih[2 * H:],
            bhh[:H], bhh[H:2 * H], bhh[2 * H:]]
    mat = pl.BlockSpec((H, H), lambda i: (0, 0))
    vec = pl.BlockSpec((H,), lambda i: (0,))
    return pl.pallas_call(
        _gru_body,
        grid=(N // blk,),
        in_specs=[pl.BlockSpec((blk, HP), lambda i: (i, 0)), vec,
                  pl.BlockSpec((blk, H), lambda i: (i, 0)),
                  mat, mat, mat, mat, mat, mat, vec, vec, vec, vec, vec, vec],
        out_specs=pl.BlockSpec((blk, H), lambda i: (i, 0)),
        out_shape=jax.ShapeDtypeStruct((N, H), jnp.float32),
    )(*args)


@_sc_lazy(
    out_type=(jax.ShapeDtypeStruct((NW * CAP,), jnp.int32),
              jax.ShapeDtypeStruct((NW * CAP,), jnp.int32),
              jax.ShapeDtypeStruct((NW * CAP,), jnp.float32),
              jax.ShapeDtypeStruct((NW * CAP,), jnp.float32),
              jax.ShapeDtypeStruct((NW * CAP,), jnp.float32),
              jax.ShapeDtypeStruct((NW * CAP,), jnp.float32),
              jax.ShapeDtypeStruct((NW, 16), jnp.int32)),
    scratch_types=[
        pltpu.VMEM((CH,), jnp.int32), pltpu.VMEM((CH,), jnp.int32),
        pltpu.VMEM((CH * EDGE_DIM,), jnp.float32),
        pltpu.VMEM((CAP + 80,), jnp.int32), pltpu.VMEM((CAP + 80,), jnp.int32),
        pltpu.VMEM((CAP + 80,), jnp.float32), pltpu.VMEM((CAP + 80,), jnp.float32),
        pltpu.VMEM((CAP + 80,), jnp.float32), pltpu.VMEM((CAP + 80,), jnp.float32),
        pltpu.VMEM((16,), jnp.int32),
    ],
)
def _k_lists(src_h, dst_h, eaf_h, sl_h, dl_h, ea0_h, ea1_h, ea2_h, ea3_h, cnt_h,
             srcc, dstc, eac, sbuf, dbuf, ea0b, ea1b, ea2b, ea3b, cntv):
    eabufs = (ea0b, ea1b, ea2b, ea3b)
    eal_hs = (ea0_h, ea1_h, ea2_h, ea3_h)
    wid = _wid()
    lo = wid * NPT
    hi = lo + NPT
    lane = lax.iota(jnp.int32, 16)

    def chunk(ci, ptr):
        base = ci * CH
        pltpu.sync_copy(src_h.at[pl.ds(base, CH)], srcc)
        pltpu.sync_copy(dst_h.at[pl.ds(base, CH)], dstc)
        pltpu.sync_copy(eaf_h.at[pl.ds(base * EDGE_DIM, CH * EDGE_DIM)], eac)

        def group(j, p):
            dv = dstc[pl.ds(j * 16, 16)]
            sv = srcc[pl.ds(j * 16, 16)]
            m = (dv >= lo) & (dv < hi)
            plsc.store_compressed(sbuf.at[pl.ds(p, 16)], sv, mask=m)
            plsc.store_compressed(dbuf.at[pl.ds(p, 16)], dv - lo, mask=m)
            eix = (j * 16 + lane) * EDGE_DIM
            for d in range(EDGE_DIM):
                ev = plsc.load_gather(eac, [eix + d])
                plsc.store_compressed(eabufs[d].at[pl.ds(p, 16)], ev, mask=m)
            return p + jnp.max(plsc.all_reduce_population_count(m))

        return lax.fori_loop(0, CH // 16, group, ptr)

    ptr = lax.fori_loop(0, E // CH, chunk, jnp.int32(0))

    zero16 = jnp.zeros((16,), jnp.int32)
    sent16 = jnp.full((16,), NPT, jnp.int32)
    zf16 = jnp.zeros((16,), jnp.float32)
    for gpad in range(KB // 16):
        sbuf[pl.ds(ptr + gpad * 16, 16)] = zero16
        dbuf[pl.ds(ptr + gpad * 16, 16)] = sent16
        for d in range(EDGE_DIM):
            eabufs[d][pl.ds(ptr + gpad * 16, 16)] = zf16

    cntv[...] = jnp.broadcast_to(ptr, (16,)).astype(jnp.int32)
    pltpu.sync_copy(sbuf.at[pl.ds(0, CAP)], sl_h.at[pl.ds(wid * CAP, CAP)])
    pltpu.sync_copy(dbuf.at[pl.ds(0, CAP)], dl_h.at[pl.ds(wid * CAP, CAP)])
    for d in range(EDGE_DIM):
        pltpu.sync_copy(eabufs[d].at[pl.ds(0, CAP)], eal_hs[d].at[pl.ds(wid * CAP, CAP)])
    pltpu.sync_copy(cntv, cnt_h.at[wid])


@_sc_lazy(
    out_type=jax.ShapeDtypeStruct((N_PAD, HP), jnp.float32),
    scratch_types=[
        pltpu.VMEM((N,), jnp.float32),
        pltpu.VMEM((NPT + 16,), jnp.float32),
        pltpu.VMEM((NPT + 16,), jnp.float32),
        pltpu.VMEM((NPT + 16,), jnp.float32),
        pltpu.VMEM((NPT + 16,), jnp.float32),
        pltpu.VMEM((NPT + 1, HP), jnp.float32),
        pltpu.VMEM((KB,), jnp.int32),
        pltpu.VMEM((KB,), jnp.int32),
        pltpu.VMEM((KB,), jnp.float32),
        pltpu.VMEM((KB, HP), jnp.float32),
        pltpu.VMEM((16,), jnp.float32),
        pltpu.VMEM((16,), jnp.int32),
        pltpu.SemaphoreType.DMA,
    ],
)
def _k_edge_gat(sl_h, dl_h, cnt_h, h_h, as_h, adp_h, asm_h, msg_h,
                astab, adl, mbl, sl, invl, acc, sbb, dbb, ebb, rows, asmv, cv, sem):
    wid = _wid()
    lo = wid * NPT
    zf = jnp.zeros((16,), jnp.float32)
    lane = lax.iota(jnp.int32, 16)

    pltpu.sync_copy(as_h, astab)
    pltpu.sync_copy(adp_h.at[pl.ds(lo, NPT)], adl.at[pl.ds(0, NPT)])
    pltpu.sync_copy(asm_h, asmv)
    pltpu.sync_copy(cnt_h.at[wid], cv)

    for j in range(NPT // 16):
        adv = adl[pl.ds(j * 16, 16)]
        mbl[pl.ds(j * 16, 16)] = _leaky(asmv[...] + adv)
        sl[pl.ds(j * 16, 16)] = zf

    def zrow(r, _):
        for j in range(HP // 16):
            acc[r, pl.ds(j * 16, 16)] = zf
        return 0
    lax.fori_loop(0, NPT + 1, zrow, 0)

    cnt = cv[pl.ds(0, 16)][0]
    nbat = (cnt + (KB - 1)) // KB

    def batch(b, _):
        off = wid * CAP + b * KB
        pltpu.sync_copy(sl_h.at[pl.ds(off, KB)], sbb)
        pltpu.sync_copy(dl_h.at[pl.ds(off, KB)], dbb)
        dma = pltpu.async_copy(h_h.at[sbb], rows, sem)

        def alpha_g(g, _2):
            s16 = pl.ds(g * 16, 16)
            sv = sbb[s16]
            dv = dbb[s16]
            asv = plsc.load_gather(astab, [sv])
            adv = plsc.load_gather(adl, [dv])
            mbv = plsc.load_gather(mbl, [dv])
            ev = jnp.exp(_leaky(asv + adv) - mbv)
            ev = jnp.where(dv >= NPT, 0.0, ev)
            ebb[s16] = ev
            plsc.addupdate_scatter(sl, [dv], ev)
            return 0
        lax.fori_loop(0, KB // 16, alpha_g, 0)
        dma.wait()

        for g in range(KB // 16):
            s16 = pl.ds(g * 16, 16)
            ev16 = ebb[s16]
            dv16 = dbb[s16]
            rowv = lane + g * 16

            @plsc.parallel_loop(0, HP, unroll=4)
            def _(h):
                hv = jnp.broadcast_to(h, (16,)).astype(jnp.int32)
                col = plsc.load_gather(rows, [rowv, hv])
                plsc.addupdate_scatter(acc, [dv16, hv], col * ev16)
        return 0

    lax.fori_loop(0, nbat, batch, 0)

    for j in range(NPT // 16):
        sv = sl[pl.ds(j * 16, 16)]
        invl[pl.ds(j * 16, 16)] = jnp.where(sv > 0, 1.0 / sv, 0.0)

    def finrow(r, _):
        inv = invl[pl.ds(r, 16)][0]
        for j in range(HP // 16):
            acc[r, pl.ds(j * 16, 16)] *= inv
        return 0
    lax.fori_loop(0, NPT, finrow, 0)
    pltpu.sync_copy(acc.at[pl.ds(0, NPT)], msg_h.at[pl.ds(lo, NPT)])


@_sc_lazy(
    out_type=jax.ShapeDtypeStruct((N_PAD, HP), jnp.float32),
    scratch_types=[
        pltpu.VMEM((NPT + 16,), jnp.float32),
        pltpu.VMEM((NPT + 16,), jnp.float32),
        pltpu.VMEM((NPT + 16,), jnp.float32),
        pltpu.VMEM((NPT + 16,), jnp.float32),
        pltpu.VMEM((NPT + 1, HP), jnp.float32),
        pltpu.VMEM((KB,), jnp.int32),
        pltpu.VMEM((KB,), jnp.int32),
        pltpu.VMEM((KB,), jnp.float32), pltpu.VMEM((KB,), jnp.float32),
        pltpu.VMEM((KB,), jnp.float32), pltpu.VMEM((KB,), jnp.float32),
        pltpu.VMEM((KB,), jnp.float32),
        pltpu.VMEM((KB, HP), jnp.float32),
        pltpu.VMEM((KB, HP), jnp.float32),
        pltpu.VMEM((HP + 16,), jnp.float32),
        pltpu.VMEM((EDGE_DIM * HP + 16,), jnp.float32),
        pltpu.VMEM((16,), jnp.float32),
        pltpu.VMEM((16,), jnp.int32),
        pltpu.SemaphoreType.DMA,
        pltpu.SemaphoreType.DMA,
    ],
)
def _k_edge_gate(sl_h, dl_h, ea0_h, ea1_h, ea2_h, ea3_h, cnt_h, u_h, w_h,
                 attl_h, w1e_h, ar_h, bd_h, msg_h,
                 arl, mbl, sl, invl, acc, sbb, dbb, ea0b, ea1b, ea2b, ea3b,
                 ebb, urows, wrows, attl, w1ec, bdv, cv, sem1, sem2):
    ea_hs = (ea0_h, ea1_h, ea2_h, ea3_h)
    eabufs = (ea0b, ea1b, ea2b, ea3b)
    wid = _wid()
    lo = wid * NPT
    zf = jnp.zeros((16,), jnp.float32)
    lane = lax.iota(jnp.int32, 16)

    pltpu.sync_copy(ar_h.at[pl.ds(lo, NPT)], arl.at[pl.ds(0, NPT)])
    pltpu.sync_copy(attl_h, attl.at[pl.ds(0, HP)])
    pltpu.sync_copy(w1e_h, w1ec.at[pl.ds(0, EDGE_DIM * HP)])
    pltpu.sync_copy(bd_h, bdv)
    pltpu.sync_copy(cnt_h.at[wid], cv)

    for j in range(NPT // 16):
        arv = arl[pl.ds(j * 16, 16)]
        mbl[pl.ds(j * 16, 16)] = _leaky(bdv[...] + arv)
        sl[pl.ds(j * 16, 16)] = zf

    def zrow(r, _):
        for j in range(HP // 16):
            acc[r, pl.ds(j * 16, 16)] = zf
        return 0
    lax.fori_loop(0, NPT + 1, zrow, 0)

    cnt = cv[pl.ds(0, 16)][0]
    nbat = (cnt + (KB - 1)) // KB

    def batch(b, _):
        off = wid * CAP + b * KB
        pltpu.sync_copy(sl_h.at[pl.ds(off, KB)], sbb)
        pltpu.sync_copy(dl_h.at[pl.ds(off, KB)], dbb)
        for d in range(EDGE_DIM):
            pltpu.sync_copy(ea_hs[d].at[pl.ds(off, KB)], eabufs[d])
        dma_u = pltpu.async_copy(u_h.at[sbb], urows, sem1)
        dma_w = pltpu.async_copy(w_h.at[sbb], wrows, sem2)
        dma_u.wait()

        eav = [[eabufs[d][pl.ds(g * 16, 16)] for d in range(EDGE_DIM)]
               for g in range(KB // 16)]
        rowvs = [lane + g * 16 for g in range(KB // 16)]

        @plsc.parallel_loop(0, HP, unroll=2, carry=(zf, zf, zf, zf))
        def dots(h, carry):
            hv = jnp.broadcast_to(h, (16,)).astype(jnp.int32)
            al_h = attl[pl.ds(h, 16)][0]
            ws = [w1ec[pl.ds(d * HP + h, 16)][0] for d in range(EDGE_DIM)]
            out = []
            for g in range(KB // 16):
                uv = plsc.load_gather(urows, [rowvs[g], hv])
                vv = eav[g][0] * ws[0]
                for d in range(1, EDGE_DIM):
                    vv += eav[g][d] * ws[d]
                out.append(carry[g] + al_h * _leaky(uv + vv))
            return tuple(out)

        for g in range(KB // 16):
            s16 = pl.ds(g * 16, 16)
            dv = dbb[s16]
            arv = plsc.load_gather(arl, [dv])
            mbv = plsc.load_gather(mbl, [dv])
            ev = jnp.exp(_leaky(dots[g] + arv) - mbv)
            ev = jnp.where(dv >= NPT, 0.0, ev)
            ebb[s16] = ev
            plsc.addupdate_scatter(sl, [dv], ev)
        dma_w.wait()

        for g in range(KB // 16):
            s16 = pl.ds(g * 16, 16)
            ev16 = ebb[s16]
            dv16 = dbb[s16]
            rowv = rowvs[g]

            @plsc.parallel_loop(0, HP, unroll=4)
            def _(h):
                hv = jnp.broadcast_to(h, (16,)).astype(jnp.int32)
                col = plsc.load_gather(wrows, [rowv, hv])
                plsc.addupdate_scatter(acc, [dv16, hv], col * ev16)
        return 0

    lax.fori_loop(0, nbat, batch, 0)

    for j in range(NPT // 16):
        sv = sl[pl.ds(j * 16, 16)]
        invl[pl.ds(j * 16, 16)] = jnp.where(sv > 0, 1.0 / sv, 0.0)

    def finrow(r, _):
        inv = invl[pl.ds(r, 16)][0]
        for j in range(HP // 16):
            acc[r, pl.ds(j * 16, 16)] *= inv
        return 0
    lax.fori_loop(0, NPT, finrow, 0)
    pltpu.sync_copy(acc.at[pl.ds(0, NPT)], msg_h.at[pl.ds(lo, NPT)])


def _molpre_body(xv_ref, lw_ref, asrc_ref, batch_ref, hs_ref, as_ref, asmax_ref, out0_ref):
    i = pl.program_id(0)
    hs = jnp.dot(xv_ref[...], lw_ref[...].T, preferred_element_type=jnp.float32)
    hs_ref[...] = hs
    a_s = jnp.sum(hs * asrc_ref[...][None, :], axis=1)
    as_ref[...] = a_s[:, None]
    oneh = (batch_ref[...]
            == jax.lax.broadcasted_iota(jnp.int32, (1000, G), 1)).astype(jnp.float32)

    @pl.when(i == 0)
    def _():
        asmax_ref[...] = jnp.full((8, 128), -1e30, jnp.float32)
        out0_ref[...] = jnp.zeros((G, H), jnp.float32)
    asmax_ref[...] = jnp.maximum(asmax_ref[...], jnp.max(a_s))
    out0_ref[...] += jax.lax.dot_general(oneh, xv_ref[...], (((0,), (0,)), ((), ())),
                                         preferred_element_type=jnp.float32)

    @pl.when(i == pl.num_programs(0) - 1)
    def _():
        out0_ref[...] = jax.nn.relu(out0_ref[...])


def _molpre(xv, lw, asrc, batch):
    blk = 1000
    return pl.pallas_call(
        _molpre_body,
        grid=(N // blk,),
        in_specs=[
            pl.BlockSpec((blk, H), lambda i: (i, 0)),
            pl.BlockSpec((H, H), lambda i: (0, 0)),
            pl.BlockSpec((H,), lambda i: (0,)),
            pl.BlockSpec((blk, 1), lambda i: (i, 0)),
        ],
        out_specs=[
            pl.BlockSpec((blk, H), lambda i: (i, 0)),
            pl.BlockSpec((blk, 1), lambda i: (i, 0)),
            pl.BlockSpec((8, 128), lambda i: (0, 0)),
            pl.BlockSpec((G, H), lambda i: (0, 0)),
        ],
        out_shape=[
            jax.ShapeDtypeStruct((N, H), jnp.float32),
            jax.ShapeDtypeStruct((N, 1), jnp.float32),
            jax.ShapeDtypeStruct((8, 128), jnp.float32),
            jax.ShapeDtypeStruct((G, H), jnp.float32),
        ],
    )(xv, lw, asrc, batch)


def _mola_body(out_ref, lw_ref, adst_ref, asmax_ref, mb_ref, ad_ref):
    hd = jnp.dot(out_ref[...], lw_ref[...].T, preferred_element_type=jnp.float32)
    a_d = jnp.sum(hd * adst_ref[...][None, :], axis=1)
    ad_ref[...] = a_d
    mb_ref[...] = _leaky(jnp.max(asmax_ref[...]) + a_d)


def _mola(out, lw, adst, asmax):
    return pl.pallas_call(
        _mola_body,
        grid=(1,),
        in_specs=[
            pl.BlockSpec((G, H), lambda i: (0, 0)),
            pl.BlockSpec((H, H), lambda i: (0, 0)),
            pl.BlockSpec((H,), lambda i: (0,)),
            pl.BlockSpec((8, 128), lambda i: (0, 0)),
        ],
        out_specs=[pl.BlockSpec((G,), lambda i: (0,)), pl.BlockSpec((G,), lambda i: (0,))],
        out_shape=[jax.ShapeDtypeStruct((G,), jnp.float32),
                   jax.ShapeDtypeStruct((G,), jnp.float32)],
    )(out, lw, adst, asmax)


def _molmain_body(hs_ref, as_ref, batch_ref, ad_ref, mb_ref, s_ref, u_ref):
    i = pl.program_id(0)
    oneh = (batch_ref[...]
            == jax.lax.broadcasted_iota(jnp.int32, (1000, G), 1)).astype(jnp.float32)
    adn = oneh @ ad_ref[...]
    mbn = oneh @ mb_ref[...]
    e = jnp.exp(_leaky(as_ref[...][:, 0] + adn) - mbn)

    @pl.when(i == 0)
    def _():
        s_ref[...] = jnp.zeros((G,), jnp.float32)
        u_ref[...] = jnp.zeros((G, H), jnp.float32)
    s_ref[...] += e @ oneh
    u_ref[...] += jax.lax.dot_general(oneh, hs_ref[...] * e[:, None],
                                      (((0,), (0,)), ((), ())),
                                      preferred_element_type=jnp.float32)


def _molmain(hs, a_s, batch, a_d, mb):
    blk = 1000
    return pl.pallas_call(
        _molmain_body,
        grid=(N // blk,),
        in_specs=[
            pl.BlockSpec((blk, H), lambda i: (i, 0)),
            pl.BlockSpec((blk, 1), lambda i: (i, 0)),
            pl.BlockSpec((blk, 1), lambda i: (i, 0)),
            pl.BlockSpec((G,), lambda i: (0,)),
            pl.BlockSpec((G,), lambda i: (0,)),
        ],
        out_specs=[pl.BlockSpec((G,), lambda i: (0,)),
                   pl.BlockSpec((G, H), lambda i: (0, 0))],
        out_shape=[jax.ShapeDtypeStruct((G,), jnp.float32),
                   jax.ShapeDtypeStruct((G, H), jnp.float32)],
    )(hs, a_s, batch, a_d, mb)


def _molfin_body(u_ref, s_ref, out_ref, bias_ref, wr_ref, wz_ref, wn_ref,
                 vr_ref, vz_ref, vn_ref, br_ref, bz_ref, bn_ref,
                 cr_ref, cz_ref, cn_ref, l2w_ref, l2b_ref, outn_ref, pred_ref):
    s = s_ref[...]
    sinv = jnp.where(s > 0, 1.0 / s, 0.0)
    msg = u_ref[...] * sinv[:, None] + bias_ref[...][None, :]
    h = _elu(msg)
    xv = out_ref[...]
    dot = lambda a, b: jnp.dot(a, b.T, preferred_element_type=jnp.float32)
    i_r = dot(h, wr_ref[...]) + br_ref[...]
    i_z = dot(h, wz_ref[...]) + bz_ref[...]
    i_n = dot(h, wn_ref[...]) + bn_ref[...]
    h_r = dot(xv, vr_ref[...]) + cr_ref[...]
    h_z = dot(xv, vz_ref[...]) + cz_ref[...]
    h_n = dot(xv, vn_ref[...]) + cn_ref[...]
    r = jax.nn.sigmoid(i_r + h_r)
    z = jax.nn.sigmoid(i_z + h_z)
    n = jnp.tanh(i_n + r * h_n)
    outn = jax.nn.relu((1.0 - z) * n + z * xv)
    outn_ref[...] = outn
    pred_ref[...] = jnp.dot(outn, l2w_ref[...].T, preferred_element_type=jnp.float32) \
        + l2b_ref[...][None, :]


def _molfin(u, s, out, bias, gru_p, l2wp, l2bp):
    wih, whh, bih, bhh = gru_p["wih"], gru_p["whh"], gru_p["bih"], gru_p["bhh"]
    args = [u, s, out, bias,
            wih[:H], wih[H:2 * H], wih[2 * H:],
            whh[:H], whh[H:2 * H], whh[2 * H:],
            bih[:H], bih[H:2 * H], bih[2 * H:],
            bhh[:H], bhh[H:2 * H], bhh[2 * H:],
            l2wp, l2bp]
    mat = pl.BlockSpec((H, H), lambda i: (0, 0))
    vec = pl.BlockSpec((H,), lambda i: (0,))
    gh = pl.BlockSpec((G, H), lambda i: (0, 0))
    return pl.pallas_call(
        _molfin_body,
        grid=(1,),
        in_specs=[gh, pl.BlockSpec((G,), lambda i: (0,)), gh, vec,
                  mat, mat, mat, mat, mat, mat, vec, vec, vec, vec, vec, vec,
                  pl.BlockSpec((8, H), lambda i: (0, 0)),
                  pl.BlockSpec((8,), lambda i: (0,))],
        out_specs=[gh, pl.BlockSpec((G, 8), lambda i: (0, 0))],
        out_shape=[jax.ShapeDtypeStruct((G, H), jnp.float32),
                   jax.ShapeDtypeStruct((G, 8), jnp.float32)],
    )(*args)


def _pad_rows(w, rows):
    return jnp.pad(w, ((0, rows - w.shape[0]), (0, 0)))


def kernel(x, edge_index, edge_attr, batch, params):
    src, dst = edge_index[0], edge_index[1]
    g = params["gate"]
    w1 = g["lin1_w"]

    slist, dlist, ea0, ea1, ea2, ea3, cnt = _k_lists(src, dst, edge_attr.reshape(-1))

    w1xp = _pad_rows(w1[:, :H], HP)
    w2p = _pad_rows(g["lin2_w"], HP)
    xv, u, w, a_r, bmax = _embed(x, params["lin1_w"], params["lin1_b"],
                                 w1xp, w2p, g["att_r"], g["att_l"], w1[:, H:])
    arp = jnp.pad(a_r.reshape(-1), (0, N_PAD - N))
    bd16 = jnp.broadcast_to(bmax[0, 0], (16,))
    attlp = jnp.pad(g["att_l"], (0, HP - H))
    w1ecp = jnp.pad(w1[:, H:].T, ((0, 0), (0, HP - H)))

    msg = _k_edge_gate(slist, dlist, ea0, ea1, ea2, ea3, cnt, u, w,
                       attlp, w1ecp.reshape(-1), arp, bd16)
    xv = _gru(msg, g["bias"], xv, params["gru0"])

    for conv_p, gru_p in zip(params["atom_conv"], params["atom_gru"]):
        lwp = _pad_rows(conv_p["lin_w"], HP)
        h, a_s, a_d, asmax = _proj(xv, lwp, conv_p["att_src"], conv_p["att_dst"])
        adp = jnp.pad(a_d.reshape(-1), (0, N_PAD - N))
        asm16 = jnp.broadcast_to(asmax[0, 0], (16,))
        msg = _k_edge_gat(slist, dlist, cnt, h, a_s.reshape(-1), adp, asm16)
        xv = _gru(msg, conv_p["bias"], xv, gru_p)

    mp = params["mol_conv"]
    batch2 = batch[:, None]
    hs, a_s, asmax, out = _molpre(xv, mp["lin_w"], mp["att_src"], batch2)
    l2wp = _pad_rows(params["lin2_w"], 8)
    l2bp = jnp.pad(params["lin2_b"], (0, 7))
    pred = None
    for _ in range(3):
        mb, a_d = _mola(out, mp["lin_w"], mp["att_dst"], asmax)
        s, uacc = _molmain(hs, a_s, batch2, a_d, mb)
        out, pred8 = _molfin(uacc, s, out, mp["bias"], params["mol_gru"], l2wp, l2bp)
        pred = pred8[:, :1]
    return pred


# speedup vs baseline: 1.7878x; 1.7878x over previous
import functools

import jax
import jax.numpy as jnp
from jax import lax
from jax.experimental import pallas as pl
from jax.experimental.pallas import tpu as pltpu
from jax.experimental.pallas import tpu_sc as plsc

N = 10000
E = 320000
IN_CH = 128
H = 200
HP = 256
EDGE_DIM = 4
G = 64
N_PAD = 10240
NW = 32
NPT = N_PAD // NW
CAP = 16384
KB = 64
CH = 2000

@functools.cache
def _mesh():
    return plsc.VectorSubcoreMesh(core_axis_name="c", subcore_axis_name="s",
                                  num_cores=2, num_subcores=16)
_sc_params = pltpu.CompilerParams(needs_layout_passes=False)


def _sc_lazy(**kw):
    def deco(body):
        @functools.cache
        def make():
            return pl.kernel(body, mesh=_mesh(), compiler_params=_sc_params, **kw)
        return lambda *args: make()(*args)
    return deco

def _leaky(v, slope=0.01):
    return jnp.where(v >= 0, v, slope * v)


def _elu(v):
    return jnp.where(v > 0, v, jnp.exp(jnp.minimum(v, 0.0)) - 1.0)

def _wid():
    return lax.axis_index("c") * 16 + lax.axis_index("s")


def _embed_body(x_ref, w1_ref, b1_ref, w1xp_ref, w2p_ref, attr_ref, attl_ref,
                w1e_ref, xv_ref, u_ref, w_ref, ar_ref, bmax_ref):
    i = pl.program_id(0)
    xv = _leaky(jnp.dot(x_ref[...], w1_ref[...].T, preferred_element_type=jnp.float32)
                + b1_ref[...])
    xv_ref[...] = xv
    u = jnp.dot(xv, w1xp_ref[...].T, preferred_element_type=jnp.float32)
    u_ref[...] = u
    w_ref[...] = jnp.dot(xv, w2p_ref[...].T, preferred_element_type=jnp.float32)
    ar_ref[...] = jnp.sum(xv * attr_ref[...][None, :], axis=1)[:, None]
    w1e = w1e_ref[...]
    vmin = jnp.sum(jnp.minimum(w1e, 0.0), axis=1)
    vmax = jnp.sum(jnp.maximum(w1e, 0.0), axis=1)
    attl = attl_ref[...]
    u200 = u[:, :H]
    tb = jnp.maximum(attl[None, :] * _leaky(u200 + vmin[None, :]),
                     attl[None, :] * _leaky(u200 + vmax[None, :]))
    bmx = jnp.max(jnp.sum(tb, axis=1))

    @pl.when(i == 0)
    def _():
        bmax_ref[...] = jnp.full((8, 128), -1e30, jnp.float32)
    bmax_ref[...] = jnp.maximum(bmax_ref[...], bmx)


def _embed(x, w1, b1, w1xp, w2p, attr, attl, w1e):
    blk = 1000
    return pl.pallas_call(
        _embed_body,
        grid=(N // blk,),
        in_specs=[
            pl.BlockSpec((blk, IN_CH), lambda i: (i, 0)),
            pl.BlockSpec((H, IN_CH), lambda i: (0, 0)),
            pl.BlockSpec((H,), lambda i: (0,)),
            pl.BlockSpec((HP, H), lambda i: (0, 0)),
            pl.BlockSpec((HP, H), lambda i: (0, 0)),
            pl.BlockSpec((H,), lambda i: (0,)),
            pl.BlockSpec((H,), lambda i: (0,)),
            pl.BlockSpec((H, EDGE_DIM), lambda i: (0, 0)),
        ],
        out_specs=[
            pl.BlockSpec((blk, H), lambda i: (i, 0)),
            pl.BlockSpec((blk, HP), lambda i: (i, 0)),
            pl.BlockSpec((blk, HP), lambda i: (i, 0)),
            pl.BlockSpec((blk, 1), lambda i: (i, 0)),
            pl.BlockSpec((8, 128), lambda i: (0, 0)),
        ],
        out_shape=[
            jax.ShapeDtypeStruct((N, H), jnp.float32),
            jax.ShapeDtypeStruct((N, HP), jnp.float32),
            jax.ShapeDtypeStruct((N, HP), jnp.float32),
            jax.ShapeDtypeStruct((N, 1), jnp.float32),
            jax.ShapeDtypeStruct((8, 128), jnp.float32),
        ],
    )(x, w1, b1, w1xp, w2p, attr, attl, w1e)


def _proj_body(xv_ref, lwp_ref, asrc_ref, adst_ref, h_ref, as_ref, ad_ref, asmax_ref):
    i = pl.program_id(0)
    h = jnp.dot(xv_ref[...], lwp_ref[...].T, preferred_element_type=jnp.float32)
    h_ref[...] = h
    h200 = h[:, :H]
    a_s = jnp.sum(h200 * asrc_ref[...][None, :], axis=1)
    as_ref[...] = a_s[:, None]
    ad_ref[...] = jnp.sum(h200 * adst_ref[...][None, :], axis=1)[:, None]

    @pl.when(i == 0)
    def _():
        asmax_ref[...] = jnp.full((8, 128), -1e30, jnp.float32)
    asmax_ref[...] = jnp.maximum(asmax_ref[...], jnp.max(a_s))


def _proj(xv, lwp, asrc, adst):
    blk = 1000
    return pl.pallas_call(
        _proj_body,
        grid=(N // blk,),
        in_specs=[
            pl.BlockSpec((blk, H), lambda i: (i, 0)),
            pl.BlockSpec((HP, H), lambda i: (0, 0)),
            pl.BlockSpec((H,), lambda i: (0,)),
            pl.BlockSpec((H,), lambda i: (0,)),
        ],
        out_specs=[
            pl.BlockSpec((blk, HP), lambda i: (i, 0)),
            pl.BlockSpec((blk, 1), lambda i: (i, 0)),
            pl.BlockSpec((blk, 1), lambda i: (i, 0)),
            pl.BlockSpec((8, 128), lambda i: (0, 0)),
        ],
        out_shape=[
            jax.ShapeDtypeStruct((N, HP), jnp.float32),
            jax.ShapeDtypeStruct((N, 1), jnp.float32),
            jax.ShapeDtypeStruct((N, 1), jnp.float32),
            jax.ShapeDtypeStruct((8, 128), jnp.float32),
        ],
    )(xv, lwp, asrc, adst)


def _gru_body(msg_ref, bias_ref, xv_ref, wr_ref, wz_ref, wn_ref,
              vr_ref, vz_ref, vn_ref, br_ref, bz_ref, bn_ref,
              cr_ref, cz_ref, cn_ref, out_ref):
    h = _elu(msg_ref[...][:, :H] + bias_ref[...][None, :])
    xv = xv_ref[...]
    dot = lambda a, b: jnp.dot(a, b.T, preferred_element_type=jnp.float32)
    i_r = dot(h, wr_ref[...]) + br_ref[...]
    i_z = dot(h, wz_ref[...]) + bz_ref[...]
    i_n = dot(h, wn_ref[...]) + bn_ref[...]
    h_r = dot(xv, vr_ref[...]) + cr_ref[...]
    h_z = dot(xv, vz_ref[...]) + cz_ref[...]
    h_n = dot(xv, vn_ref[...]) + cn_ref[...]
    r = jax.nn.sigmoid(i_r + h_r)
    z = jax.nn.sigmoid(i_z + h_z)
    n = jnp.tanh(i_n + r * h_n)
    out_ref[...] = jax.nn.relu((1.0 - z) * n + z * xv)


def _gru(msg, bias, xv, gru_p):
    blk = 1000
    wih, whh, bih, bhh = gru_p["wih"], gru_p["whh"], gru_p["bih"], gru_p["bhh"]
    args = [msg, bias, xv,
            wih[:H], wih[H:2 * H], wih[2 * H:],
            whh[:H], whh[H:2 * H], whh[2 * H:],
            bih[:H], bih[H:2 * H], bih[2 * H:],
            bhh[:H], bhh[H:2 * H], bhh[2 * H:]]
    mat = pl.BlockSpec((H, H), lambda i: (0, 0))
    vec = pl.BlockSpec((H,), lambda i: (0,))
    return pl.pallas_call(
        _gru_body,
        grid=(N // blk,),
        in_specs=[pl.BlockSpec((blk, HP), lambda i: (i, 0)), vec,
                  pl.BlockSpec((blk, H), lambda i: (i, 0)),
                  mat, mat, mat, mat, mat, mat, vec, vec, vec, vec, vec, vec],
        out_specs=pl.BlockSpec((blk, H), lambda i: (i, 0)),
        out_shape=jax.ShapeDtypeStruct((N, H), jnp.float32),
    )(*args)


@_sc_lazy(
    out_type=(jax.ShapeDtypeStruct((NW * CAP,), jnp.int32),
              jax.ShapeDtypeStruct((NW * CAP,), jnp.int32),
              jax.ShapeDtypeStruct((NW * CAP,), jnp.float32),
              jax.ShapeDtypeStruct((NW * CAP,), jnp.float32),
              jax.ShapeDtypeStruct((NW * CAP,), jnp.float32),
              jax.ShapeDtypeStruct((NW * CAP,), jnp.float32),
              jax.ShapeDtypeStruct((NW, 16), jnp.int32)),
    scratch_types=[
        pltpu.VMEM((CH,), jnp.int32), pltpu.VMEM((CH,), jnp.int32),
        pltpu.VMEM((CH * EDGE_DIM,), jnp.float32),
        pltpu.VMEM((CAP + 80,), jnp.int32), pltpu.VMEM((CAP + 80,), jnp.int32),
        pltpu.VMEM((CAP + 80,), jnp.float32), pltpu.VMEM((CAP + 80,), jnp.float32),
        pltpu.VMEM((CAP + 80,), jnp.float32), pltpu.VMEM((CAP + 80,), jnp.float32),
        pltpu.VMEM((16,), jnp.int32),
    ],
)
def _k_lists(src_h, dst_h, eaf_h, sl_h, dl_h, ea0_h, ea1_h, ea2_h, ea3_h, cnt_h,
             srcc, dstc, eac, sbuf, dbuf, ea0b, ea1b, ea2b, ea3b, cntv):
    eabufs = (ea0b, ea1b, ea2b, ea3b)
    eal_hs = (ea0_h, ea1_h, ea2_h, ea3_h)
    wid = _wid()
    lo = wid * NPT
    hi = lo + NPT
    lane = lax.iota(jnp.int32, 16)

    def chunk(ci, ptr):
        base = ci * CH
        pltpu.sync_copy(src_h.at[pl.ds(base, CH)], srcc)
        pltpu.sync_copy(dst_h.at[pl.ds(base, CH)], dstc)
        pltpu.sync_copy(eaf_h.at[pl.ds(base * EDGE_DIM, CH * EDGE_DIM)], eac)

        def group(j, p):
            dv = dstc[pl.ds(j * 16, 16)]
            sv = srcc[pl.ds(j * 16, 16)]
            m = (dv >= lo) & (dv < hi)
            plsc.store_compressed(sbuf.at[pl.ds(p, 16)], sv, mask=m)
            plsc.store_compressed(dbuf.at[pl.ds(p, 16)], dv - lo, mask=m)
            eix = (j * 16 + lane) * EDGE_DIM
            for d in range(EDGE_DIM):
                ev = plsc.load_gather(eac, [eix + d])
                plsc.store_compressed(eabufs[d].at[pl.ds(p, 16)], ev, mask=m)
            return p + jnp.max(plsc.all_reduce_population_count(m))

        return lax.fori_loop(0, CH // 16, group, ptr)

    ptr = lax.fori_loop(0, E // CH, chunk, jnp.int32(0))

    zero16 = jnp.zeros((16,), jnp.int32)
    sent16 = jnp.full((16,), NPT, jnp.int32)
    zf16 = jnp.zeros((16,), jnp.float32)
    for gpad in range(KB // 16):
        sbuf[pl.ds(ptr + gpad * 16, 16)] = zero16
        dbuf[pl.ds(ptr + gpad * 16, 16)] = sent16
        for d in range(EDGE_DIM):
            eabufs[d][pl.ds(ptr + gpad * 16, 16)] = zf16

    cntv[...] = jnp.broadcast_to(ptr, (16,)).astype(jnp.int32)
    pltpu.sync_copy(sbuf.at[pl.ds(0, CAP)], sl_h.at[pl.ds(wid * CAP, CAP)])
    pltpu.sync_copy(dbuf.at[pl.ds(0, CAP)], dl_h.at[pl.ds(wid * CAP, CAP)])
    for d in range(EDGE_DIM):
        pltpu.sync_copy(eabufs[d].at[pl.ds(0, CAP)], eal_hs[d].at[pl.ds(wid * CAP, CAP)])
    pltpu.sync_copy(cntv, cnt_h.at[wid])


@_sc_lazy(
    out_type=jax.ShapeDtypeStruct((N_PAD, HP), jnp.float32),
    scratch_types=[
        pltpu.VMEM((N,), jnp.float32),
        pltpu.VMEM((NPT + 16,), jnp.float32),
        pltpu.VMEM((NPT + 16,), jnp.float32),
        pltpu.VMEM((NPT + 16,), jnp.float32),
        pltpu.VMEM((NPT + 16,), jnp.float32),
        pltpu.VMEM((NPT + 1, HP), jnp.float32),
        pltpu.VMEM((KB,), jnp.int32),
        pltpu.VMEM((KB,), jnp.int32),
        pltpu.VMEM((KB,), jnp.float32),
        pltpu.VMEM((KB, HP), jnp.float32),
        pltpu.VMEM((16,), jnp.float32),
        pltpu.VMEM((16,), jnp.int32),
        pltpu.SemaphoreType.DMA,
    ],
)
def _k_edge_gat(sl_h, dl_h, cnt_h, h_h, as_h, adp_h, asm_h, msg_h,
                astab, adl, mbl, sl, invl, acc, sbb, dbb, ebb, rows, asmv, cv, sem):
    wid = _wid()
    lo = wid * NPT
    zf = jnp.zeros((16,), jnp.float32)
    lane = lax.iota(jnp.int32, 16)

    pltpu.sync_copy(as_h, astab)
    pltpu.sync_copy(adp_h.at[pl.ds(lo, NPT)], adl.at[pl.ds(0, NPT)])
    pltpu.sync_copy(asm_h, asmv)
    pltpu.sync_copy(cnt_h.at[wid], cv)

    for j in range(NPT // 16):
        adv = adl[pl.ds(j * 16, 16)]
        mbl[pl.ds(j * 16, 16)] = _leaky(asmv[...] + adv)
        sl[pl.ds(j * 16, 16)] = zf

    def zrow(r, _):
        for j in range(HP // 16):
            acc[r, pl.ds(j * 16, 16)] = zf
        return 0
    lax.fori_loop(0, NPT + 1, zrow, 0)

    cnt = cv[pl.ds(0, 16)][0]
    nbat = (cnt + (KB - 1)) // KB

    def batch(b, _):
        off = wid * CAP + b * KB
        pltpu.sync_copy(sl_h.at[pl.ds(off, KB)], sbb)
        pltpu.sync_copy(dl_h.at[pl.ds(off, KB)], dbb)
        dma = pltpu.async_copy(h_h.at[sbb], rows, sem)

        def alpha_g(g, _2):
            s16 = pl.ds(g * 16, 16)
            sv = sbb[s16]
            dv = dbb[s16]
            asv = plsc.load_gather(astab, [sv])
            adv = plsc.load_gather(adl, [dv])
            mbv = plsc.load_gather(mbl, [dv])
            ev = jnp.exp(_leaky(asv + adv) - mbv)
            ev = jnp.where(dv >= NPT, 0.0, ev)
            ebb[s16] = ev
            plsc.addupdate_scatter(sl, [dv], ev)
            return 0
        lax.fori_loop(0, KB // 16, alpha_g, 0)
        dma.wait()

        def accum_g(g, _2):
            s16 = pl.ds(g * 16, 16)
            ev16 = ebb[s16]
            dv16 = dbb[s16]
            for l in range(16):
                e = ev16[l]
                d = dv16[l]
                for j in range(HP // 16):
                    sj = pl.ds(j * 16, 16)
                    plsc.addupdate(acc.at[d, sj], rows[g * 16 + l, sj] * e)
            return 0
        lax.fori_loop(0, KB // 16, accum_g, 0)
        return 0

    lax.fori_loop(0, nbat, batch, 0)

    for j in range(NPT // 16):
        sv = sl[pl.ds(j * 16, 16)]
        invl[pl.ds(j * 16, 16)] = jnp.where(sv > 0, 1.0 / sv, 0.0)

    def finrow(r, _):
        inv = invl[pl.ds(r, 16)][0]
        for j in range(HP // 16):
            acc[r, pl.ds(j * 16, 16)] *= inv
        return 0
    lax.fori_loop(0, NPT, finrow, 0)
    pltpu.sync_copy(acc.at[pl.ds(0, NPT)], msg_h.at[pl.ds(lo, NPT)])


@_sc_lazy(
    out_type=jax.ShapeDtypeStruct((N_PAD, HP), jnp.float32),
    scratch_types=[
        pltpu.VMEM((NPT + 16,), jnp.float32),
        pltpu.VMEM((NPT + 16,), jnp.float32),
        pltpu.VMEM((NPT + 16,), jnp.float32),
        pltpu.VMEM((NPT + 16,), jnp.float32),
        pltpu.VMEM((NPT + 1, HP), jnp.float32),
        pltpu.VMEM((KB,), jnp.int32),
        pltpu.VMEM((KB,), jnp.int32),
        pltpu.VMEM((KB,), jnp.float32), pltpu.VMEM((KB,), jnp.float32),
        pltpu.VMEM((KB,), jnp.float32), pltpu.VMEM((KB,), jnp.float32),
        pltpu.VMEM((KB,), jnp.float32),
        pltpu.VMEM((KB, HP), jnp.float32),
        pltpu.VMEM((KB, HP), jnp.float32),
        pltpu.VMEM((HP + 16,), jnp.float32),
        pltpu.VMEM((EDGE_DIM * HP + 16,), jnp.float32),
        pltpu.VMEM((16,), jnp.float32),
        pltpu.VMEM((16,), jnp.int32),
        pltpu.SemaphoreType.DMA,
        pltpu.SemaphoreType.DMA,
    ],
)
def _k_edge_gate(sl_h, dl_h, ea0_h, ea1_h, ea2_h, ea3_h, cnt_h, u_h, w_h,
                 attl_h, w1e_h, ar_h, bd_h, msg_h,
                 arl, mbl, sl, invl, acc, sbb, dbb, ea0b, ea1b, ea2b, ea3b,
                 ebb, urows, wrows, attl, w1ec, bdv, cv, sem1, sem2):
    ea_hs = (ea0_h, ea1_h, ea2_h, ea3_h)
    eabufs = (ea0b, ea1b, ea2b, ea3b)
    wid = _wid()
    lo = wid * NPT
    zf = jnp.zeros((16,), jnp.float32)
    lane = lax.iota(jnp.int32, 16)

    pltpu.sync_copy(ar_h.at[pl.ds(lo, NPT)], arl.at[pl.ds(0, NPT)])
    pltpu.sync_copy(attl_h, attl.at[pl.ds(0, HP)])
    pltpu.sync_copy(w1e_h, w1ec.at[pl.ds(0, EDGE_DIM * HP)])
    pltpu.sync_copy(bd_h, bdv)
    pltpu.sync_copy(cnt_h.at[wid], cv)

    for j in range(NPT // 16):
        arv = arl[pl.ds(j * 16, 16)]
        mbl[pl.ds(j * 16, 16)] = _leaky(bdv[...] + arv)
        sl[pl.ds(j * 16, 16)] = zf

    def zrow(r, _):
        for j in range(HP // 16):
            acc[r, pl.ds(j * 16, 16)] = zf
        return 0
    lax.fori_loop(0, NPT + 1, zrow, 0)

    cnt = cv[pl.ds(0, 16)][0]
    nbat = (cnt + (KB - 1)) // KB

    def batch(b, _):
        off = wid * CAP + b * KB
        pltpu.sync_copy(sl_h.at[pl.ds(off, KB)], sbb)
        pltpu.sync_copy(dl_h.at[pl.ds(off, KB)], dbb)
        for d in range(EDGE_DIM):
            pltpu.sync_copy(ea_hs[d].at[pl.ds(off, KB)], eabufs[d])
        dma_u = pltpu.async_copy(u_h.at[sbb], urows, sem1)
        dma_w = pltpu.async_copy(w_h.at[sbb], wrows, sem2)
        dma_u.wait()

        eav = [[eabufs[d][pl.ds(g * 16, 16)] for d in range(EDGE_DIM)]
               for g in range(KB // 16)]
        rowvs = [lane + g * 16 for g in range(KB // 16)]

        @plsc.parallel_loop(0, HP, unroll=2, carry=(zf, zf, zf, zf))
        def dots(h, carry):
            hv = jnp.broadcast_to(h, (16,)).astype(jnp.int32)
            al_h = attl[pl.ds(h, 16)][0]
            ws = [w1ec[pl.ds(d * HP + h, 16)][0] for d in range(EDGE_DIM)]
            out = []
            for g in range(KB // 16):
                uv = plsc.load_gather(urows, [rowvs[g], hv])
                vv = eav[g][0] * ws[0]
                for d in range(1, EDGE_DIM):
                    vv += eav[g][d] * ws[d]
                out.append(carry[g] + al_h * _leaky(uv + vv))
            return tuple(out)

        for g in range(KB // 16):
            s16 = pl.ds(g * 16, 16)
            dv = dbb[s16]
            arv = plsc.load_gather(arl, [dv])
            mbv = plsc.load_gather(mbl, [dv])
            ev = jnp.exp(_leaky(dots[g] + arv) - mbv)
            ev = jnp.where(dv >= NPT, 0.0, ev)
            ebb[s16] = ev
            plsc.addupdate_scatter(sl, [dv], ev)
        dma_w.wait()

        def accum_g(g, _2):
            s16 = pl.ds(g * 16, 16)
            ev16 = ebb[s16]
            dv16 = dbb[s16]
            for l in range(16):
                e = ev16[l]
                d = dv16[l]
                for j in range(HP // 16):
                    sj = pl.ds(j * 16, 16)
                    plsc.addupdate(acc.at[d, sj], wrows[g * 16 + l, sj] * e)
            return 0
        lax.fori_loop(0, KB // 16, accum_g, 0)
        return 0

    lax.fori_loop(0, nbat, batch, 0)

    for j in range(NPT // 16):
        sv = sl[pl.ds(j * 16, 16)]
        invl[pl.ds(j * 16, 16)] = jnp.where(sv > 0, 1.0 / sv, 0.0)

    def finrow(r, _):
        inv = invl[pl.ds(r, 16)][0]
        for j in range(HP // 16):
            acc[r, pl.ds(j * 16, 16)] *= inv
        return 0
    lax.fori_loop(0, NPT, finrow, 0)
    pltpu.sync_copy(acc.at[pl.ds(0, NPT)], msg_h.at[pl.ds(lo, NPT)])


def _molpre_body(xv_ref, lw_ref, asrc_ref, batch_ref, hs_ref, as_ref, asmax_ref, out0_ref):
    i = pl.program_id(0)
    hs = jnp.dot(xv_ref[...], lw_ref[...].T, preferred_element_type=jnp.float32)
    hs_ref[...] = hs
    a_s = jnp.sum(hs * asrc_ref[...][None, :], axis=1)
    as_ref[...] = a_s[:, None]
    oneh = (batch_ref[...]
            == jax.lax.broadcasted_iota(jnp.int32, (1000, G), 1)).astype(jnp.float32)

    @pl.when(i == 0)
    def _():
        asmax_ref[...] = jnp.full((8, 128), -1e30, jnp.float32)
        out0_ref[...] = jnp.zeros((G, H), jnp.float32)
    asmax_ref[...] = jnp.maximum(asmax_ref[...], jnp.max(a_s))
    out0_ref[...] += jax.lax.dot_general(oneh, xv_ref[...], (((0,), (0,)), ((), ())),
                                         preferred_element_type=jnp.float32)

    @pl.when(i == pl.num_programs(0) - 1)
    def _():
        out0_ref[...] = jax.nn.relu(out0_ref[...])


def _molpre(xv, lw, asrc, batch):
    blk = 1000
    return pl.pallas_call(
        _molpre_body,
        grid=(N // blk,),
        in_specs=[
            pl.BlockSpec((blk, H), lambda i: (i, 0)),
            pl.BlockSpec((H, H), lambda i: (0, 0)),
            pl.BlockSpec((H,), lambda i: (0,)),
            pl.BlockSpec((blk, 1), lambda i: (i, 0)),
        ],
        out_specs=[
            pl.BlockSpec((blk, H), lambda i: (i, 0)),
            pl.BlockSpec((blk, 1), lambda i: (i, 0)),
            pl.BlockSpec((8, 128), lambda i: (0, 0)),
            pl.BlockSpec((G, H), lambda i: (0, 0)),
        ],
        out_shape=[
            jax.ShapeDtypeStruct((N, H), jnp.float32),
            jax.ShapeDtypeStruct((N, 1), jnp.float32),
            jax.ShapeDtypeStruct((8, 128), jnp.float32),
            jax.ShapeDtypeStruct((G, H), jnp.float32),
        ],
    )(xv, lw, asrc, batch)


def _mola_body(out_ref, lw_ref, adst_ref, asmax_ref, mb_ref, ad_ref):
    hd = jnp.dot(out_ref[...], lw_ref[...].T, preferred_element_type=jnp.float32)
    a_d = jnp.sum(hd * adst_ref[...][None, :], axis=1)
    ad_ref[...] = a_d
    mb_ref[...] = _leaky(jnp.max(asmax_ref[...]) + a_d)


def _mola(out, lw, adst, asmax):
    return pl.pallas_call(
        _mola_body,
        grid=(1,),
        in_specs=[
            pl.BlockSpec((G, H), lambda i: (0, 0)),
            pl.BlockSpec((H, H), lambda i: (0, 0)),
            pl.BlockSpec((H,), lambda i: (0,)),
            pl.BlockSpec((8, 128), lambda i: (0, 0)),
        ],
        out_specs=[pl.BlockSpec((G,), lambda i: (0,)), pl.BlockSpec((G,), lambda i: (0,))],
        out_shape=[jax.ShapeDtypeStruct((G,), jnp.float32),
                   jax.ShapeDtypeStruct((G,), jnp.float32)],
    )(out, lw, adst, asmax)


def _molmain_body(hs_ref, as_ref, batch_ref, ad_ref, mb_ref, s_ref, u_ref):
    i = pl.program_id(0)
    oneh = (batch_ref[...]
            == jax.lax.broadcasted_iota(jnp.int32, (1000, G), 1)).astype(jnp.float32)
    adn = oneh @ ad_ref[...]
    mbn = oneh @ mb_ref[...]
    e = jnp.exp(_leaky(as_ref[...][:, 0] + adn) - mbn)

    @pl.when(i == 0)
    def _():
        s_ref[...] = jnp.zeros((G,), jnp.float32)
        u_ref[...] = jnp.zeros((G, H), jnp.float32)
    s_ref[...] += e @ oneh
    u_ref[...] += jax.lax.dot_general(oneh, hs_ref[...] * e[:, None],
                                      (((0,), (0,)), ((), ())),
                                      preferred_element_type=jnp.float32)


def _molmain(hs, a_s, batch, a_d, mb):
    blk = 1000
    return pl.pallas_call(
        _molmain_body,
        grid=(N // blk,),
        in_specs=[
            pl.BlockSpec((blk, H), lambda i: (i, 0)),
            pl.BlockSpec((blk, 1), lambda i: (i, 0)),
            pl.BlockSpec((blk, 1), lambda i: (i, 0)),
            pl.BlockSpec((G,), lambda i: (0,)),
            pl.BlockSpec((G,), lambda i: (0,)),
        ],
        out_specs=[pl.BlockSpec((G,), lambda i: (0,)),
                   pl.BlockSpec((G, H), lambda i: (0, 0))],
        out_shape=[jax.ShapeDtypeStruct((G,), jnp.float32),
                   jax.ShapeDtypeStruct((G, H), jnp.float32)],
    )(hs, a_s, batch, a_d, mb)


def _molfin_body(u_ref, s_ref, out_ref, bias_ref, wr_ref, wz_ref, wn_ref,
                 vr_ref, vz_ref, vn_ref, br_ref, bz_ref, bn_ref,
                 cr_ref, cz_ref, cn_ref, l2w_ref, l2b_ref, outn_ref, pred_ref):
    s = s_ref[...]
    sinv = jnp.where(s > 0, 1.0 / s, 0.0)
    msg = u_ref[...] * sinv[:, None] + bias_ref[...][None, :]
    h = _elu(msg)
    xv = out_ref[...]
    dot = lambda a, b: jnp.dot(a, b.T, preferred_element_type=jnp.float32)
    i_r = dot(h, wr_ref[...]) + br_ref[...]
    i_z = dot(h, wz_ref[...]) + bz_ref[...]
    i_n = dot(h, wn_ref[...]) + bn_ref[...]
    h_r = dot(xv, vr_ref[...]) + cr_ref[...]
    h_z = dot(xv, vz_ref[...]) + cz_ref[...]
    h_n = dot(xv, vn_ref[...]) + cn_ref[...]
    r = jax.nn.sigmoid(i_r + h_r)
    z = jax.nn.sigmoid(i_z + h_z)
    n = jnp.tanh(i_n + r * h_n)
    outn = jax.nn.relu((1.0 - z) * n + z * xv)
    outn_ref[...] = outn
    pred_ref[...] = jnp.dot(outn, l2w_ref[...].T, preferred_element_type=jnp.float32) \
        + l2b_ref[...][None, :]


def _molfin(u, s, out, bias, gru_p, l2wp, l2bp):
    wih, whh, bih, bhh = gru_p["wih"], gru_p["whh"], gru_p["bih"], gru_p["bhh"]
    args = [u, s, out, bias,
            wih[:H], wih[H:2 * H], wih[2 * H:],
            whh[:H], whh[H:2 * H], whh[2 * H:],
            bih[:H], bih[H:2 * H], bih[2 * H:],
            bhh[:H], bhh[H:2 * H], bhh[2 * H:],
            l2wp, l2bp]
    mat = pl.BlockSpec((H, H), lambda i: (0, 0))
    vec = pl.BlockSpec((H,), lambda i: (0,))
    gh = pl.BlockSpec((G, H), lambda i: (0, 0))
    return pl.pallas_call(
        _molfin_body,
        grid=(1,),
        in_specs=[gh, pl.BlockSpec((G,), lambda i: (0,)), gh, vec,
                  mat, mat, mat, mat, mat, mat, vec, vec, vec, vec, vec, vec,
                  pl.BlockSpec((8, H), lambda i: (0, 0)),
                  pl.BlockSpec((8,), lambda i: (0,))],
        out_specs=[gh, pl.BlockSpec((G, 8), lambda i: (0, 0))],
        out_shape=[jax.ShapeDtypeStruct((G, H), jnp.float32),
                   jax.ShapeDtypeStruct((G, 8), jnp.float32)],
    )(*args)


def _pad_rows(w, rows):
    return jnp.pad(w, ((0, rows - w.shape[0]), (0, 0)))


def kernel(x, edge_index, edge_attr, batch, params):
    src, dst = edge_index[0], edge_index[1]
    g = params["gate"]
    w1 = g["lin1_w"]

    slist, dlist, ea0, ea1, ea2, ea3, cnt = _k_lists(src, dst, edge_attr.reshape(-1))

    w1xp = _pad_rows(w1[:, :H], HP)
    w2p = _pad_rows(g["lin2_w"], HP)
    xv, u, w, a_r, bmax = _embed(x, params["lin1_w"], params["lin1_b"],
                                 w1xp, w2p, g["att_r"], g["att_l"], w1[:, H:])
    arp = jnp.pad(a_r.reshape(-1), (0, N_PAD - N))
    bd16 = jnp.broadcast_to(bmax[0, 0], (16,))
    attlp = jnp.pad(g["att_l"], (0, HP - H))
    w1ecp = jnp.pad(w1[:, H:].T, ((0, 0), (0, HP - H)))

    msg = _k_edge_gate(slist, dlist, ea0, ea1, ea2, ea3, cnt, u, w,
                       attlp, w1ecp.reshape(-1), arp, bd16)
    xv = _gru(msg, g["bias"], xv, params["gru0"])

    for conv_p, gru_p in zip(params["atom_conv"], params["atom_gru"]):
        lwp = _pad_rows(conv_p["lin_w"], HP)
        h, a_s, a_d, asmax = _proj(xv, lwp, conv_p["att_src"], conv_p["att_dst"])
        adp = jnp.pad(a_d.reshape(-1), (0, N_PAD - N))
        asm16 = jnp.broadcast_to(asmax[0, 0], (16,))
        msg = _k_edge_gat(slist, dlist, cnt, h, a_s.reshape(-1), adp, asm16)
        xv = _gru(msg, conv_p["bias"], xv, gru_p)

    mp = params["mol_conv"]
    batch2 = batch[:, None]
    hs, a_s, asmax, out = _molpre(xv, mp["lin_w"], mp["att_src"], batch2)
    l2wp = _pad_rows(params["lin2_w"], 8)
    l2bp = jnp.pad(params["lin2_b"], (0, 7))
    pred = None
    for _ in range(3):
        mb, a_d = _mola(out, mp["lin_w"], mp["att_dst"], asmax)
        s, uacc = _molmain(hs, a_s, batch2, a_d, mb)
        out, pred8 = _molfin(uacc, s, out, mp["bias"], params["mol_gru"], l2wp, l2bp)
        pred = pred8[:, :1]
    return pred


# speedup vs baseline: 2.0294x; 1.1352x over previous
import functools

import jax
import jax.numpy as jnp
from jax import lax
from jax.experimental import pallas as pl
from jax.experimental.pallas import tpu as pltpu
from jax.experimental.pallas import tpu_sc as plsc

N = 10000
E = 320000
IN_CH = 128
H = 200
HP = 256
EDGE_DIM = 4
G = 64
N_PAD = 10240
NW = 32
NPT = N_PAD // NW
CAP = 16384
KB = 64
CH = 2000

@functools.cache
def _mesh():
    return plsc.VectorSubcoreMesh(core_axis_name="c", subcore_axis_name="s",
                                  num_cores=2, num_subcores=16)
_sc_params = pltpu.CompilerParams(needs_layout_passes=False)


def _sc_lazy(**kw):
    def deco(body):
        @functools.cache
        def make():
            return pl.kernel(body, mesh=_mesh(), compiler_params=_sc_params, **kw)
        return lambda *args: make()(*args)
    return deco

def _leaky(v, slope=0.01):
    return jnp.where(v >= 0, v, slope * v)


def _elu(v):
    return jnp.where(v > 0, v, jnp.exp(jnp.minimum(v, 0.0)) - 1.0)

def _wid():
    return lax.axis_index("c") * 16 + lax.axis_index("s")


def _embed_body(x_ref, w1_ref, b1_ref, w1xp_ref, w2p_ref, attr_ref, attl_ref,
                w1e_ref, xv_ref, u_ref, w_ref, ar_ref, bmax_ref):
    i = pl.program_id(0)
    xv = _leaky(jnp.dot(x_ref[...], w1_ref[...].T, preferred_element_type=jnp.float32)
                + b1_ref[...])
    xv_ref[...] = xv
    u = jnp.dot(xv, w1xp_ref[...].T, preferred_element_type=jnp.float32)
    u_ref[...] = u
    w_ref[...] = jnp.dot(xv, w2p_ref[...].T, preferred_element_type=jnp.float32)
    ar_ref[...] = jnp.sum(xv * attr_ref[...][None, :], axis=1)[:, None]
    w1e = w1e_ref[...]
    vmin = jnp.sum(jnp.minimum(w1e, 0.0), axis=1)
    vmax = jnp.sum(jnp.maximum(w1e, 0.0), axis=1)
    attl = attl_ref[...]
    u200 = u[:, :H]
    tb = jnp.maximum(attl[None, :] * _leaky(u200 + vmin[None, :]),
                     attl[None, :] * _leaky(u200 + vmax[None, :]))
    bmx = jnp.max(jnp.sum(tb, axis=1))

    @pl.when(i == 0)
    def _():
        bmax_ref[...] = jnp.full((8, 128), -1e30, jnp.float32)
    bmax_ref[...] = jnp.maximum(bmax_ref[...], bmx)


def _embed(x, w1, b1, w1xp, w2p, attr, attl, w1e):
    blk = 1000
    return pl.pallas_call(
        _embed_body,
        grid=(N // blk,),
        in_specs=[
            pl.BlockSpec((blk, IN_CH), lambda i: (i, 0)),
            pl.BlockSpec((H, IN_CH), lambda i: (0, 0)),
            pl.BlockSpec((H,), lambda i: (0,)),
            pl.BlockSpec((HP, H), lambda i: (0, 0)),
            pl.BlockSpec((HP, H), lambda i: (0, 0)),
            pl.BlockSpec((H,), lambda i: (0,)),
            pl.BlockSpec((H,), lambda i: (0,)),
            pl.BlockSpec((H, EDGE_DIM), lambda i: (0, 0)),
        ],
        out_specs=[
            pl.BlockSpec((blk, H), lambda i: (i, 0)),
            pl.BlockSpec((blk, HP), lambda i: (i, 0)),
            pl.BlockSpec((blk, HP), lambda i: (i, 0)),
            pl.BlockSpec((blk, 1), lambda i: (i, 0)),
            pl.BlockSpec((8, 128), lambda i: (0, 0)),
        ],
        out_shape=[
            jax.ShapeDtypeStruct((N, H), jnp.float32),
            jax.ShapeDtypeStruct((N, HP), jnp.float32),
            jax.ShapeDtypeStruct((N, HP), jnp.float32),
            jax.ShapeDtypeStruct((N, 1), jnp.float32),
            jax.ShapeDtypeStruct((8, 128), jnp.float32),
        ],
    )(x, w1, b1, w1xp, w2p, attr, attl, w1e)


def _proj_body(xv_ref, lwp_ref, asrc_ref, adst_ref, h_ref, as_ref, ad_ref, asmax_ref):
    i = pl.program_id(0)
    h = jnp.dot(xv_ref[...], lwp_ref[...].T, preferred_element_type=jnp.float32)
    h_ref[...] = h
    h200 = h[:, :H]
    a_s = jnp.sum(h200 * asrc_ref[...][None, :], axis=1)
    as_ref[...] = a_s[:, None]
    ad_ref[...] = jnp.sum(h200 * adst_ref[...][None, :], axis=1)[:, None]

    @pl.when(i == 0)
    def _():
        asmax_ref[...] = jnp.full((8, 128), -1e30, jnp.float32)
    asmax_ref[...] = jnp.maximum(asmax_ref[...], jnp.max(a_s))


def _proj(xv, lwp, asrc, adst):
    blk = 1000
    return pl.pallas_call(
        _proj_body,
        grid=(N // blk,),
        in_specs=[
            pl.BlockSpec((blk, H), lambda i: (i, 0)),
            pl.BlockSpec((HP, H), lambda i: (0, 0)),
            pl.BlockSpec((H,), lambda i: (0,)),
            pl.BlockSpec((H,), lambda i: (0,)),
        ],
        out_specs=[
            pl.BlockSpec((blk, HP), lambda i: (i, 0)),
            pl.BlockSpec((blk, 1), lambda i: (i, 0)),
            pl.BlockSpec((blk, 1), lambda i: (i, 0)),
            pl.BlockSpec((8, 128), lambda i: (0, 0)),
        ],
        out_shape=[
            jax.ShapeDtypeStruct((N, HP), jnp.float32),
            jax.ShapeDtypeStruct((N, 1), jnp.float32),
            jax.ShapeDtypeStruct((N, 1), jnp.float32),
            jax.ShapeDtypeStruct((8, 128), jnp.float32),
        ],
    )(xv, lwp, asrc, adst)


def _gru_body(msg_ref, bias_ref, xv_ref, wr_ref, wz_ref, wn_ref,
              vr_ref, vz_ref, vn_ref, br_ref, bz_ref, bn_ref,
              cr_ref, cz_ref, cn_ref, out_ref):
    h = _elu(msg_ref[...][:, :H] + bias_ref[...][None, :])
    xv = xv_ref[...]
    dot = lambda a, b: jnp.dot(a, b.T, preferred_element_type=jnp.float32)
    i_r = dot(h, wr_ref[...]) + br_ref[...]
    i_z = dot(h, wz_ref[...]) + bz_ref[...]
    i_n = dot(h, wn_ref[...]) + bn_ref[...]
    h_r = dot(xv, vr_ref[...]) + cr_ref[...]
    h_z = dot(xv, vz_ref[...]) + cz_ref[...]
    h_n = dot(xv, vn_ref[...]) + cn_ref[...]
    r = jax.nn.sigmoid(i_r + h_r)
    z = jax.nn.sigmoid(i_z + h_z)
    n = jnp.tanh(i_n + r * h_n)
    out_ref[...] = jax.nn.relu((1.0 - z) * n + z * xv)


def _gru(msg, bias, xv, gru_p):
    blk = 1000
    wih, whh, bih, bhh = gru_p["wih"], gru_p["whh"], gru_p["bih"], gru_p["bhh"]
    args = [msg, bias, xv,
            wih[:H], wih[H:2 * H], wih[2 * H:],
            whh[:H], whh[H:2 * H], whh[2 * H:],
            bih[:H], bih[H:2 * H], bih[2 * H:],
            bhh[:H], bhh[H:2 * H], bhh[2 * H:]]
    mat = pl.BlockSpec((H, H), lambda i: (0, 0))
    vec = pl.BlockSpec((H,), lambda i: (0,))
    return pl.pallas_call(
        _gru_body,
        grid=(N // blk,),
        in_specs=[pl.BlockSpec((blk, HP), lambda i: (i, 0)), vec,
                  pl.BlockSpec((blk, H), lambda i: (i, 0)),
                  mat, mat, mat, mat, mat, mat, vec, vec, vec, vec, vec, vec],
        out_specs=pl.BlockSpec((blk, H), lambda i: (i, 0)),
        out_shape=jax.ShapeDtypeStruct((N, H), jnp.float32),
    )(*args)


@_sc_lazy(
    out_type=(jax.ShapeDtypeStruct((NW * CAP,), jnp.int32),
              jax.ShapeDtypeStruct((NW * CAP,), jnp.int32),
              jax.ShapeDtypeStruct((NW * CAP,), jnp.float32),
              jax.ShapeDtypeStruct((NW * CAP,), jnp.float32),
              jax.ShapeDtypeStruct((NW * CAP,), jnp.float32),
              jax.ShapeDtypeStruct((NW * CAP,), jnp.float32),
              jax.ShapeDtypeStruct((NW, 16), jnp.int32)),
    scratch_types=[
        pltpu.VMEM((CH,), jnp.int32), pltpu.VMEM((CH,), jnp.int32),
        pltpu.VMEM((CH * EDGE_DIM,), jnp.float32),
        pltpu.VMEM((CAP + 80,), jnp.int32), pltpu.VMEM((CAP + 80,), jnp.int32),
        pltpu.VMEM((CAP + 80,), jnp.float32), pltpu.VMEM((CAP + 80,), jnp.float32),
        pltpu.VMEM((CAP + 80,), jnp.float32), pltpu.VMEM((CAP + 80,), jnp.float32),
        pltpu.VMEM((16,), jnp.int32),
    ],
)
def _k_lists(src_h, dst_h, eaf_h, sl_h, dl_h, ea0_h, ea1_h, ea2_h, ea3_h, cnt_h,
             srcc, dstc, eac, sbuf, dbuf, ea0b, ea1b, ea2b, ea3b, cntv):
    eabufs = (ea0b, ea1b, ea2b, ea3b)
    eal_hs = (ea0_h, ea1_h, ea2_h, ea3_h)
    wid = _wid()
    lo = wid * NPT
    hi = lo + NPT
    lane = lax.iota(jnp.int32, 16)

    def chunk(ci, ptr):
        base = ci * CH
        pltpu.sync_copy(src_h.at[pl.ds(base, CH)], srcc)
        pltpu.sync_copy(dst_h.at[pl.ds(base, CH)], dstc)
        pltpu.sync_copy(eaf_h.at[pl.ds(base * EDGE_DIM, CH * EDGE_DIM)], eac)

        def group(j, p):
            dv = dstc[pl.ds(j * 16, 16)]
            sv = srcc[pl.ds(j * 16, 16)]
            m = (dv >= lo) & (dv < hi)
            plsc.store_compressed(sbuf.at[pl.ds(p, 16)], sv, mask=m)
            plsc.store_compressed(dbuf.at[pl.ds(p, 16)], dv - lo, mask=m)
            eix = (j * 16 + lane) * EDGE_DIM
            for d in range(EDGE_DIM):
                ev = plsc.load_gather(eac, [eix + d])
                plsc.store_compressed(eabufs[d].at[pl.ds(p, 16)], ev, mask=m)
            return p + jnp.max(plsc.all_reduce_population_count(m))

        return lax.fori_loop(0, CH // 16, group, ptr)

    ptr = lax.fori_loop(0, E // CH, chunk, jnp.int32(0))

    zero16 = jnp.zeros((16,), jnp.int32)
    sent16 = jnp.full((16,), NPT, jnp.int32)
    zf16 = jnp.zeros((16,), jnp.float32)
    for gpad in range(KB // 16):
        sbuf[pl.ds(ptr + gpad * 16, 16)] = zero16
        dbuf[pl.ds(ptr + gpad * 16, 16)] = sent16
        for d in range(EDGE_DIM):
            eabufs[d][pl.ds(ptr + gpad * 16, 16)] = zf16

    cntv[...] = jnp.broadcast_to(ptr, (16,)).astype(jnp.int32)
    pltpu.sync_copy(sbuf.at[pl.ds(0, CAP)], sl_h.at[pl.ds(wid * CAP, CAP)])
    pltpu.sync_copy(dbuf.at[pl.ds(0, CAP)], dl_h.at[pl.ds(wid * CAP, CAP)])
    for d in range(EDGE_DIM):
        pltpu.sync_copy(eabufs[d].at[pl.ds(0, CAP)], eal_hs[d].at[pl.ds(wid * CAP, CAP)])
    pltpu.sync_copy(cntv, cnt_h.at[wid])


@_sc_lazy(
    out_type=jax.ShapeDtypeStruct((N_PAD, HP), jnp.float32),
    scratch_types=[
        pltpu.VMEM((N,), jnp.float32),
        pltpu.VMEM((NPT + 16,), jnp.float32),
        pltpu.VMEM((NPT + 16,), jnp.float32),
        pltpu.VMEM((NPT + 16,), jnp.float32),
        pltpu.VMEM((NPT + 16,), jnp.float32),
        pltpu.VMEM((NPT + 1, HP), jnp.float32),
        pltpu.VMEM((KB,), jnp.int32),
        pltpu.VMEM((KB,), jnp.int32),
        pltpu.VMEM((KB,), jnp.float32),
        pltpu.VMEM((KB, HP), jnp.float32),
        pltpu.VMEM((16,), jnp.float32),
        pltpu.VMEM((16,), jnp.int32),
        pltpu.SemaphoreType.DMA,
    ],
)
def _k_edge_gat(sl_h, dl_h, cnt_h, h_h, as_h, adp_h, asm_h, msg_h,
                astab, adl, mbl, sl, invl, acc, sbb, dbb, ebb, rows, asmv, cv, sem):
    wid = _wid()
    lo = wid * NPT
    zf = jnp.zeros((16,), jnp.float32)
    lane = lax.iota(jnp.int32, 16)

    pltpu.sync_copy(as_h, astab)
    pltpu.sync_copy(adp_h.at[pl.ds(lo, NPT)], adl.at[pl.ds(0, NPT)])
    pltpu.sync_copy(asm_h, asmv)
    pltpu.sync_copy(cnt_h.at[wid], cv)

    for j in range(NPT // 16):
        adv = adl[pl.ds(j * 16, 16)]
        mbl[pl.ds(j * 16, 16)] = _leaky(asmv[...] + adv)
        sl[pl.ds(j * 16, 16)] = zf

    def zrow(r, _):
        for j in range(HP // 16):
            acc[r, pl.ds(j * 16, 16)] = zf
        return 0
    lax.fori_loop(0, NPT + 1, zrow, 0)

    cnt = cv[pl.ds(0, 16)][0]
    nbat = (cnt + (KB - 1)) // KB

    def batch(b, _):
        off = wid * CAP + b * KB
        pltpu.sync_copy(sl_h.at[pl.ds(off, KB)], sbb)
        pltpu.sync_copy(dl_h.at[pl.ds(off, KB)], dbb)
        dma = pltpu.async_copy(h_h.at[sbb], rows, sem)

        def alpha_g(g, _2):
            s16 = pl.ds(g * 16, 16)
            sv = sbb[s16]
            dv = dbb[s16]
            asv = plsc.load_gather(astab, [sv])
            adv = plsc.load_gather(adl, [dv])
            mbv = plsc.load_gather(mbl, [dv])
            ev = jnp.exp(_leaky(asv + adv) - mbv)
            ev = jnp.where(dv >= NPT, 0.0, ev)
            ebb[s16] = ev
            plsc.addupdate_scatter(sl, [dv], ev)
            return 0
        lax.fori_loop(0, KB // 16, alpha_g, 0)
        dma.wait()

        def accum_g(g, _2):
            s16 = pl.ds(g * 16, 16)
            ev16 = ebb[s16]
            dv16 = dbb[s16]
            for l in range(16):
                e = ev16[l]
                d = dv16[l]
                for j in range(13):
                    sj = pl.ds(j * 16, 16)
                    plsc.addupdate(acc.at[d, sj], rows[g * 16 + l, sj] * e)
            return 0
        lax.fori_loop(0, KB // 16, accum_g, 0)
        return 0

    lax.fori_loop(0, nbat, batch, 0)

    for j in range(NPT // 16):
        sv = sl[pl.ds(j * 16, 16)]
        invl[pl.ds(j * 16, 16)] = jnp.where(sv > 0, 1.0 / sv, 0.0)

    def finrow(r, _):
        inv = invl[pl.ds(r, 16)][0]
        for j in range(13):
            acc[r, pl.ds(j * 16, 16)] *= inv
        return 0
    lax.fori_loop(0, NPT, finrow, 0)
    pltpu.sync_copy(acc.at[pl.ds(0, NPT)], msg_h.at[pl.ds(lo, NPT)])


@_sc_lazy(
    out_type=jax.ShapeDtypeStruct((N_PAD, HP), jnp.float32),
    scratch_types=[
        pltpu.VMEM((NPT + 16,), jnp.float32),
        pltpu.VMEM((NPT + 16,), jnp.float32),
        pltpu.VMEM((NPT + 16,), jnp.float32),
        pltpu.VMEM((NPT + 16,), jnp.float32),
        pltpu.VMEM((NPT + 1, HP), jnp.float32),
        pltpu.VMEM((KB,), jnp.int32),
        pltpu.VMEM((KB,), jnp.int32),
        pltpu.VMEM((KB,), jnp.float32), pltpu.VMEM((KB,), jnp.float32),
        pltpu.VMEM((KB,), jnp.float32), pltpu.VMEM((KB,), jnp.float32),
        pltpu.VMEM((KB,), jnp.float32),
        pltpu.VMEM((KB, HP), jnp.float32),
        pltpu.VMEM((KB, HP), jnp.float32),
        pltpu.VMEM((HP + 16,), jnp.float32),
        pltpu.VMEM((EDGE_DIM * HP + 16,), jnp.float32),
        pltpu.VMEM((16,), jnp.float32),
        pltpu.VMEM((16,), jnp.int32),
        pltpu.SemaphoreType.DMA,
        pltpu.SemaphoreType.DMA,
    ],
)
def _k_edge_gate(sl_h, dl_h, ea0_h, ea1_h, ea2_h, ea3_h, cnt_h, u_h, w_h,
                 attl_h, w1e_h, ar_h, bd_h, msg_h,
                 arl, mbl, sl, invl, acc, sbb, dbb, ea0b, ea1b, ea2b, ea3b,
                 ebb, urows, wrows, attl, w1ec, bdv, cv, sem1, sem2):
    ea_hs = (ea0_h, ea1_h, ea2_h, ea3_h)
    eabufs = (ea0b, ea1b, ea2b, ea3b)
    wid = _wid()
    lo = wid * NPT
    zf = jnp.zeros((16,), jnp.float32)
    lane = lax.iota(jnp.int32, 16)

    pltpu.sync_copy(ar_h.at[pl.ds(lo, NPT)], arl.at[pl.ds(0, NPT)])
    pltpu.sync_copy(attl_h, attl.at[pl.ds(0, HP)])
    pltpu.sync_copy(w1e_h, w1ec.at[pl.ds(0, EDGE_DIM * HP)])
    pltpu.sync_copy(bd_h, bdv)
    pltpu.sync_copy(cnt_h.at[wid], cv)

    for j in range(NPT // 16):
        arv = arl[pl.ds(j * 16, 16)]
        mbl[pl.ds(j * 16, 16)] = _leaky(bdv[...] + arv)
        sl[pl.ds(j * 16, 16)] = zf

    def zrow(r, _):
        for j in range(HP // 16):
            acc[r, pl.ds(j * 16, 16)] = zf
        return 0
    lax.fori_loop(0, NPT + 1, zrow, 0)

    cnt = cv[pl.ds(0, 16)][0]
    nbat = (cnt + (KB - 1)) // KB

    def batch(b, _):
        off = wid * CAP + b * KB
        pltpu.sync_copy(sl_h.at[pl.ds(off, KB)], sbb)
        pltpu.sync_copy(dl_h.at[pl.ds(off, KB)], dbb)
        for d in range(EDGE_DIM):
            pltpu.sync_copy(ea_hs[d].at[pl.ds(off, KB)], eabufs[d])
        dma_u = pltpu.async_copy(u_h.at[sbb], urows, sem1)
        dma_w = pltpu.async_copy(w_h.at[sbb], wrows, sem2)
        dma_u.wait()

        eav = [[eabufs[d][pl.ds(g * 16, 16)] for d in range(EDGE_DIM)]
               for g in range(KB // 16)]
        rowvs = [lane + g * 16 for g in range(KB // 16)]

        @plsc.parallel_loop(0, H, unroll=2, carry=(zf, zf, zf, zf))
        def dots(h, carry):
            hv = jnp.broadcast_to(h, (16,)).astype(jnp.int32)
            al_h = attl[pl.ds(h, 16)][0]
            ws = [w1ec[pl.ds(d * HP + h, 16)][0] for d in range(EDGE_DIM)]
            out = []
            for g in range(KB // 16):
                uv = plsc.load_gather(urows, [rowvs[g], hv])
                vv = eav[g][0] * ws[0]
                for d in range(1, EDGE_DIM):
                    vv += eav[g][d] * ws[d]
                out.append(carry[g] + al_h * _leaky(uv + vv))
            return tuple(out)

        for g in range(KB // 16):
            s16 = pl.ds(g * 16, 16)
            dv = dbb[s16]
            arv = plsc.load_gather(arl, [dv])
            mbv = plsc.load_gather(mbl, [dv])
            ev = jnp.exp(_leaky(dots[g] + arv) - mbv)
            ev = jnp.where(dv >= NPT, 0.0, ev)
            ebb[s16] = ev
            plsc.addupdate_scatter(sl, [dv], ev)
        dma_w.wait()

        def accum_g(g, _2):
            s16 = pl.ds(g * 16, 16)
            ev16 = ebb[s16]
            dv16 = dbb[s16]
            for l in range(16):
                e = ev16[l]
                d = dv16[l]
                for j in range(13):
                    sj = pl.ds(j * 16, 16)
                    plsc.addupdate(acc.at[d, sj], wrows[g * 16 + l, sj] * e)
            return 0
        lax.fori_loop(0, KB // 16, accum_g, 0)
        return 0

    lax.fori_loop(0, nbat, batch, 0)

    for j in range(NPT // 16):
        sv = sl[pl.ds(j * 16, 16)]
        invl[pl.ds(j * 16, 16)] = jnp.where(sv > 0, 1.0 / sv, 0.0)

    def finrow(r, _):
        inv = invl[pl.ds(r, 16)][0]
        for j in range(13):
            acc[r, pl.ds(j * 16, 16)] *= inv
        return 0
    lax.fori_loop(0, NPT, finrow, 0)
    pltpu.sync_copy(acc.at[pl.ds(0, NPT)], msg_h.at[pl.ds(lo, NPT)])


def _molpre_body(xv_ref, lw_ref, asrc_ref, batch_ref, hs_ref, as_ref, asmax_ref, out0_ref):
    i = pl.program_id(0)
    hs = jnp.dot(xv_ref[...], lw_ref[...].T, preferred_element_type=jnp.float32)
    hs_ref[...] = hs
    a_s = jnp.sum(hs * asrc_ref[...][None, :], axis=1)
    as_ref[...] = a_s[:, None]
    oneh = (batch_ref[...]
            == jax.lax.broadcasted_iota(jnp.int32, (1000, G), 1)).astype(jnp.float32)

    @pl.when(i == 0)
    def _():
        asmax_ref[...] = jnp.full((8, 128), -1e30, jnp.float32)
        out0_ref[...] = jnp.zeros((G, H), jnp.float32)
    asmax_ref[...] = jnp.maximum(asmax_ref[...], jnp.max(a_s))
    out0_ref[...] += jax.lax.dot_general(oneh, xv_ref[...], (((0,), (0,)), ((), ())),
                                         preferred_element_type=jnp.float32)

    @pl.when(i == pl.num_programs(0) - 1)
    def _():
        out0_ref[...] = jax.nn.relu(out0_ref[...])


def _molpre(xv, lw, asrc, batch):
    blk = 1000
    return pl.pallas_call(
        _molpre_body,
        grid=(N // blk,),
        in_specs=[
            pl.BlockSpec((blk, H), lambda i: (i, 0)),
            pl.BlockSpec((H, H), lambda i: (0, 0)),
            pl.BlockSpec((H,), lambda i: (0,)),
            pl.BlockSpec((blk, 1), lambda i: (i, 0)),
        ],
        out_specs=[
            pl.BlockSpec((blk, H), lambda i: (i, 0)),
            pl.BlockSpec((blk, 1), lambda i: (i, 0)),
            pl.BlockSpec((8, 128), lambda i: (0, 0)),
            pl.BlockSpec((G, H), lambda i: (0, 0)),
        ],
        out_shape=[
            jax.ShapeDtypeStruct((N, H), jnp.float32),
            jax.ShapeDtypeStruct((N, 1), jnp.float32),
            jax.ShapeDtypeStruct((8, 128), jnp.float32),
            jax.ShapeDtypeStruct((G, H), jnp.float32),
        ],
    )(xv, lw, asrc, batch)


def _mola_body(out_ref, lw_ref, adst_ref, asmax_ref, mb_ref, ad_ref):
    hd = jnp.dot(out_ref[...], lw_ref[...].T, preferred_element_type=jnp.float32)
    a_d = jnp.sum(hd * adst_ref[...][None, :], axis=1)
    ad_ref[...] = a_d
    mb_ref[...] = _leaky(jnp.max(asmax_ref[...]) + a_d)


def _mola(out, lw, adst, asmax):
    return pl.pallas_call(
        _mola_body,
        grid=(1,),
        in_specs=[
            pl.BlockSpec((G, H), lambda i: (0, 0)),
            pl.BlockSpec((H, H), lambda i: (0, 0)),
            pl.BlockSpec((H,), lambda i: (0,)),
            pl.BlockSpec((8, 128), lambda i: (0, 0)),
        ],
        out_specs=[pl.BlockSpec((G,), lambda i: (0,)), pl.BlockSpec((G,), lambda i: (0,))],
        out_shape=[jax.ShapeDtypeStruct((G,), jnp.float32),
                   jax.ShapeDtypeStruct((G,), jnp.float32)],
    )(out, lw, adst, asmax)


def _molmain_body(hs_ref, as_ref, batch_ref, ad_ref, mb_ref, s_ref, u_ref):
    i = pl.program_id(0)
    oneh = (batch_ref[...]
            == jax.lax.broadcasted_iota(jnp.int32, (1000, G), 1)).astype(jnp.float32)
    adn = oneh @ ad_ref[...]
    mbn = oneh @ mb_ref[...]
    e = jnp.exp(_leaky(as_ref[...][:, 0] + adn) - mbn)

    @pl.when(i == 0)
    def _():
        s_ref[...] = jnp.zeros((G,), jnp.float32)
        u_ref[...] = jnp.zeros((G, H), jnp.float32)
    s_ref[...] += e @ oneh
    u_ref[...] += jax.lax.dot_general(oneh, hs_ref[...] * e[:, None],
                                      (((0,), (0,)), ((), ())),
                                      preferred_element_type=jnp.float32)


def _molmain(hs, a_s, batch, a_d, mb):
    blk = 1000
    return pl.pallas_call(
        _molmain_body,
        grid=(N // blk,),
        in_specs=[
            pl.BlockSpec((blk, H), lambda i: (i, 0)),
            pl.BlockSpec((blk, 1), lambda i: (i, 0)),
            pl.BlockSpec((blk, 1), lambda i: (i, 0)),
            pl.BlockSpec((G,), lambda i: (0,)),
            pl.BlockSpec((G,), lambda i: (0,)),
        ],
        out_specs=[pl.BlockSpec((G,), lambda i: (0,)),
                   pl.BlockSpec((G, H), lambda i: (0, 0))],
        out_shape=[jax.ShapeDtypeStruct((G,), jnp.float32),
                   jax.ShapeDtypeStruct((G, H), jnp.float32)],
    )(hs, a_s, batch, a_d, mb)


def _molfin_body(u_ref, s_ref, out_ref, bias_ref, wr_ref, wz_ref, wn_ref,
                 vr_ref, vz_ref, vn_ref, br_ref, bz_ref, bn_ref,
                 cr_ref, cz_ref, cn_ref, l2w_ref, l2b_ref, outn_ref, pred_ref):
    s = s_ref[...]
    sinv = jnp.where(s > 0, 1.0 / s, 0.0)
    msg = u_ref[...] * sinv[:, None] + bias_ref[...][None, :]
    h = _elu(msg)
    xv = out_ref[...]
    dot = lambda a, b: jnp.dot(a, b.T, preferred_element_type=jnp.float32)
    i_r = dot(h, wr_ref[...]) + br_ref[...]
    i_z = dot(h, wz_ref[...]) + bz_ref[...]
    i_n = dot(h, wn_ref[...]) + bn_ref[...]
    h_r = dot(xv, vr_ref[...]) + cr_ref[...]
    h_z = dot(xv, vz_ref[...]) + cz_ref[...]
    h_n = dot(xv, vn_ref[...]) + cn_ref[...]
    r = jax.nn.sigmoid(i_r + h_r)
    z = jax.nn.sigmoid(i_z + h_z)
    n = jnp.tanh(i_n + r * h_n)
    outn = jax.nn.relu((1.0 - z) * n + z * xv)
    outn_ref[...] = outn
    pred_ref[...] = jnp.dot(outn, l2w_ref[...].T, preferred_element_type=jnp.float32) \
        + l2b_ref[...][None, :]


def _molfin(u, s, out, bias, gru_p, l2wp, l2bp):
    wih, whh, bih, bhh = gru_p["wih"], gru_p["whh"], gru_p["bih"], gru_p["bhh"]
    args = [u, s, out, bias,
            wih[:H], wih[H:2 * H], wih[2 * H:],
            whh[:H], whh[H:2 * H], whh[2 * H:],
            bih[:H], bih[H:2 * H], bih[2 * H:],
            bhh[:H], bhh[H:2 * H], bhh[2 * H:],
            l2wp, l2bp]
    mat = pl.BlockSpec((H, H), lambda i: (0, 0))
    vec = pl.BlockSpec((H,), lambda i: (0,))
    gh = pl.BlockSpec((G, H), lambda i: (0, 0))
    return pl.pallas_call(
        _molfin_body,
        grid=(1,),
        in_specs=[gh, pl.BlockSpec((G,), lambda i: (0,)), gh, vec,
                  mat, mat, mat, mat, mat, mat, vec, vec, vec, vec, vec, vec,
                  pl.BlockSpec((8, H), lambda i: (0, 0)),
                  pl.BlockSpec((8,), lambda i: (0,))],
        out_specs=[gh, pl.BlockSpec((G, 8), lambda i: (0, 0))],
        out_shape=[jax.ShapeDtypeStruct((G, H), jnp.float32),
                   jax.ShapeDtypeStruct((G, 8), jnp.float32)],
    )(*args)


def _pad_rows(w, rows):
    return jnp.pad(w, ((0, rows - w.shape[0]), (0, 0)))


def kernel(x, edge_index, edge_attr, batch, params):
    src, dst = edge_index[0], edge_index[1]
    g = params["gate"]
    w1 = g["lin1_w"]

    slist, dlist, ea0, ea1, ea2, ea3, cnt = _k_lists(src, dst, edge_attr.reshape(-1))

    w1xp = _pad_rows(w1[:, :H], HP)
    w2p = _pad_rows(g["lin2_w"], HP)
    xv, u, w, a_r, bmax = _embed(x, params["lin1_w"], params["lin1_b"],
                                 w1xp, w2p, g["att_r"], g["att_l"], w1[:, H:])
    arp = jnp.pad(a_r.reshape(-1), (0, N_PAD - N))
    bd16 = jnp.broadcast_to(bmax[0, 0], (16,))
    attlp = jnp.pad(g["att_l"], (0, HP - H))
    w1ecp = jnp.pad(w1[:, H:].T, ((0, 0), (0, HP - H)))

    msg = _k_edge_gate(slist, dlist, ea0, ea1, ea2, ea3, cnt, u, w,
                       attlp, w1ecp.reshape(-1), arp, bd16)
    xv = _gru(msg, g["bias"], xv, params["gru0"])

    for conv_p, gru_p in zip(params["atom_conv"], params["atom_gru"]):
        lwp = _pad_rows(conv_p["lin_w"], HP)
        h, a_s, a_d, asmax = _proj(xv, lwp, conv_p["att_src"], conv_p["att_dst"])
        adp = jnp.pad(a_d.reshape(-1), (0, N_PAD - N))
        asm16 = jnp.broadcast_to(asmax[0, 0], (16,))
        msg = _k_edge_gat(slist, dlist, cnt, h, a_s.reshape(-1), adp, asm16)
        xv = _gru(msg, conv_p["bias"], xv, gru_p)

    mp = params["mol_conv"]
    batch2 = batch[:, None]
    hs, a_s, asmax, out = _molpre(xv, mp["lin_w"], mp["att_src"], batch2)
    l2wp = _pad_rows(params["lin2_w"], 8)
    l2bp = jnp.pad(params["lin2_b"], (0, 7))
    pred = None
    for _ in range(3):
        mb, a_d = _mola(out, mp["lin_w"], mp["att_dst"], asmax)
        s, uacc = _molmain(hs, a_s, batch2, a_d, mb)
        out, pred8 = _molfin(uacc, s, out, mp["bias"], params["mol_gru"], l2wp, l2bp)
        pred = pred8[:, :1]
    return pred


# speedup vs baseline: 2.1196x; 1.0444x over previous
import functools

import jax
import jax.numpy as jnp
from jax import lax
from jax.experimental import pallas as pl
from jax.experimental.pallas import tpu as pltpu
from jax.experimental.pallas import tpu_sc as plsc

N = 10000
E = 320000
IN_CH = 128
H = 200
HP = 256
EDGE_DIM = 4
G = 64
N_PAD = 10240
NW = 32
NPT = N_PAD // NW
CAP = 16384
KB = 64
CH = 2000

@functools.cache
def _mesh():
    return plsc.VectorSubcoreMesh(core_axis_name="c", subcore_axis_name="s",
                                  num_cores=2, num_subcores=16)
_sc_params = pltpu.CompilerParams(needs_layout_passes=False)


def _sc_lazy(**kw):
    def deco(body):
        @functools.cache
        def make():
            return pl.kernel(body, mesh=_mesh(), compiler_params=_sc_params, **kw)
        return lambda *args: make()(*args)
    return deco

def _leaky(v, slope=0.01):
    return jnp.where(v >= 0, v, slope * v)


def _elu(v):
    return jnp.where(v > 0, v, jnp.exp(jnp.minimum(v, 0.0)) - 1.0)

def _wid():
    return lax.axis_index("c") * 16 + lax.axis_index("s")


def _embed_body(x_ref, w1_ref, b1_ref, w1xp_ref, w2p_ref, attr_ref, attl_ref,
                w1e_ref, xv_ref, u_ref, w_ref, ar_ref, bmax_ref):
    i = pl.program_id(0)
    xv = _leaky(jnp.dot(x_ref[...], w1_ref[...].T, preferred_element_type=jnp.float32)
                + b1_ref[...])
    xv_ref[...] = xv
    u = jnp.dot(xv, w1xp_ref[...].T, preferred_element_type=jnp.float32)
    u_ref[...] = u
    w_ref[...] = jnp.dot(xv, w2p_ref[...].T, preferred_element_type=jnp.float32)
    ar_ref[...] = jnp.sum(xv * attr_ref[...][None, :], axis=1)[:, None]
    w1e = w1e_ref[...]
    vmin = jnp.sum(jnp.minimum(w1e, 0.0), axis=1)
    vmax = jnp.sum(jnp.maximum(w1e, 0.0), axis=1)
    attl = attl_ref[...]
    u200 = u[:, :H]
    tb = jnp.maximum(attl[None, :] * _leaky(u200 + vmin[None, :]),
                     attl[None, :] * _leaky(u200 + vmax[None, :]))
    bmx = jnp.max(jnp.sum(tb, axis=1))

    @pl.when(i == 0)
    def _():
        bmax_ref[...] = jnp.full((8, 128), -1e30, jnp.float32)
    bmax_ref[...] = jnp.maximum(bmax_ref[...], bmx)


def _embed(x, w1, b1, w1xp, w2p, attr, attl, w1e):
    blk = 1000
    return pl.pallas_call(
        _embed_body,
        grid=(N // blk,),
        in_specs=[
            pl.BlockSpec((blk, IN_CH), lambda i: (i, 0)),
            pl.BlockSpec((H, IN_CH), lambda i: (0, 0)),
            pl.BlockSpec((H,), lambda i: (0,)),
            pl.BlockSpec((HP, H), lambda i: (0, 0)),
            pl.BlockSpec((HP, H), lambda i: (0, 0)),
            pl.BlockSpec((H,), lambda i: (0,)),
            pl.BlockSpec((H,), lambda i: (0,)),
            pl.BlockSpec((H, EDGE_DIM), lambda i: (0, 0)),
        ],
        out_specs=[
            pl.BlockSpec((blk, H), lambda i: (i, 0)),
            pl.BlockSpec((blk, HP), lambda i: (i, 0)),
            pl.BlockSpec((blk, HP), lambda i: (i, 0)),
            pl.BlockSpec((blk, 1), lambda i: (i, 0)),
            pl.BlockSpec((8, 128), lambda i: (0, 0)),
        ],
        out_shape=[
            jax.ShapeDtypeStruct((N, H), jnp.float32),
            jax.ShapeDtypeStruct((N, HP), jnp.float32),
            jax.ShapeDtypeStruct((N, HP), jnp.float32),
            jax.ShapeDtypeStruct((N, 1), jnp.float32),
            jax.ShapeDtypeStruct((8, 128), jnp.float32),
        ],
    )(x, w1, b1, w1xp, w2p, attr, attl, w1e)


def _proj_body(xv_ref, lwp_ref, asrc_ref, adst_ref, h_ref, as_ref, ad_ref, asmax_ref):
    i = pl.program_id(0)
    h = jnp.dot(xv_ref[...], lwp_ref[...].T, preferred_element_type=jnp.float32)
    h_ref[...] = h
    h200 = h[:, :H]
    a_s = jnp.sum(h200 * asrc_ref[...][None, :], axis=1)
    as_ref[...] = a_s[:, None]
    ad_ref[...] = jnp.sum(h200 * adst_ref[...][None, :], axis=1)[:, None]

    @pl.when(i == 0)
    def _():
        asmax_ref[...] = jnp.full((8, 128), -1e30, jnp.float32)
    asmax_ref[...] = jnp.maximum(asmax_ref[...], jnp.max(a_s))


def _proj(xv, lwp, asrc, adst):
    blk = 1000
    return pl.pallas_call(
        _proj_body,
        grid=(N // blk,),
        in_specs=[
            pl.BlockSpec((blk, H), lambda i: (i, 0)),
            pl.BlockSpec((HP, H), lambda i: (0, 0)),
            pl.BlockSpec((H,), lambda i: (0,)),
            pl.BlockSpec((H,), lambda i: (0,)),
        ],
        out_specs=[
            pl.BlockSpec((blk, HP), lambda i: (i, 0)),
            pl.BlockSpec((blk, 1), lambda i: (i, 0)),
            pl.BlockSpec((blk, 1), lambda i: (i, 0)),
            pl.BlockSpec((8, 128), lambda i: (0, 0)),
        ],
        out_shape=[
            jax.ShapeDtypeStruct((N, HP), jnp.float32),
            jax.ShapeDtypeStruct((N, 1), jnp.float32),
            jax.ShapeDtypeStruct((N, 1), jnp.float32),
            jax.ShapeDtypeStruct((8, 128), jnp.float32),
        ],
    )(xv, lwp, asrc, adst)


def _gru_body(msg_ref, bias_ref, xv_ref, wr_ref, wz_ref, wn_ref,
              vr_ref, vz_ref, vn_ref, br_ref, bz_ref, bn_ref,
              cr_ref, cz_ref, cn_ref, out_ref):
    h = _elu(msg_ref[...][:, :H] + bias_ref[...][None, :])
    xv = xv_ref[...]
    dot = lambda a, b: jnp.dot(a, b.T, preferred_element_type=jnp.float32)
    i_r = dot(h, wr_ref[...]) + br_ref[...]
    i_z = dot(h, wz_ref[...]) + bz_ref[...]
    i_n = dot(h, wn_ref[...]) + bn_ref[...]
    h_r = dot(xv, vr_ref[...]) + cr_ref[...]
    h_z = dot(xv, vz_ref[...]) + cz_ref[...]
    h_n = dot(xv, vn_ref[...]) + cn_ref[...]
    r = jax.nn.sigmoid(i_r + h_r)
    z = jax.nn.sigmoid(i_z + h_z)
    n = jnp.tanh(i_n + r * h_n)
    out_ref[...] = jax.nn.relu((1.0 - z) * n + z * xv)


def _gru(msg, bias, xv, gru_p):
    blk = 1000
    wih, whh, bih, bhh = gru_p["wih"], gru_p["whh"], gru_p["bih"], gru_p["bhh"]
    args = [msg, bias, xv,
            wih[:H], wih[H:2 * H], wih[2 * H:],
            whh[:H], whh[H:2 * H], whh[2 * H:],
            bih[:H], bih[H:2 * H], bih[2 * H:],
            bhh[:H], bhh[H:2 * H], bhh[2 * H:]]
    mat = pl.BlockSpec((H, H), lambda i: (0, 0))
    vec = pl.BlockSpec((H,), lambda i: (0,))
    return pl.pallas_call(
        _gru_body,
        grid=(N // blk,),
        in_specs=[pl.BlockSpec((blk, HP), lambda i: (i, 0)), vec,
                  pl.BlockSpec((blk, H), lambda i: (i, 0)),
                  mat, mat, mat, mat, mat, mat, vec, vec, vec, vec, vec, vec],
        out_specs=pl.BlockSpec((blk, H), lambda i: (i, 0)),
        out_shape=jax.ShapeDtypeStruct((N, H), jnp.float32),
    )(*args)


@_sc_lazy(
    out_type=(jax.ShapeDtypeStruct((NW * CAP,), jnp.int32),
              jax.ShapeDtypeStruct((NW * CAP,), jnp.int32),
              jax.ShapeDtypeStruct((NW * CAP,), jnp.float32),
              jax.ShapeDtypeStruct((NW * CAP,), jnp.float32),
              jax.ShapeDtypeStruct((NW * CAP,), jnp.float32),
              jax.ShapeDtypeStruct((NW * CAP,), jnp.float32),
              jax.ShapeDtypeStruct((NW, 16), jnp.int32)),
    scratch_types=[
        pltpu.VMEM((CH,), jnp.int32), pltpu.VMEM((CH,), jnp.int32),
        pltpu.VMEM((CH * EDGE_DIM,), jnp.float32),
        pltpu.VMEM((CAP + 80,), jnp.int32), pltpu.VMEM((CAP + 80,), jnp.int32),
        pltpu.VMEM((CAP + 80,), jnp.float32), pltpu.VMEM((CAP + 80,), jnp.float32),
        pltpu.VMEM((CAP + 80,), jnp.float32), pltpu.VMEM((CAP + 80,), jnp.float32),
        pltpu.VMEM((16,), jnp.int32),
    ],
)
def _k_lists(src_h, dst_h, eaf_h, sl_h, dl_h, ea0_h, ea1_h, ea2_h, ea3_h, cnt_h,
             srcc, dstc, eac, sbuf, dbuf, ea0b, ea1b, ea2b, ea3b, cntv):
    eabufs = (ea0b, ea1b, ea2b, ea3b)
    eal_hs = (ea0_h, ea1_h, ea2_h, ea3_h)
    wid = _wid()
    lo = wid * NPT
    hi = lo + NPT
    lane = lax.iota(jnp.int32, 16)

    def chunk(ci, ptr):
        base = ci * CH
        pltpu.sync_copy(src_h.at[pl.ds(base, CH)], srcc)
        pltpu.sync_copy(dst_h.at[pl.ds(base, CH)], dstc)
        pltpu.sync_copy(eaf_h.at[pl.ds(base * EDGE_DIM, CH * EDGE_DIM)], eac)

        def group(j, p):
            dv = dstc[pl.ds(j * 16, 16)]
            sv = srcc[pl.ds(j * 16, 16)]
            m = (dv >= lo) & (dv < hi)
            plsc.store_compressed(sbuf.at[pl.ds(p, 16)], sv, mask=m)
            plsc.store_compressed(dbuf.at[pl.ds(p, 16)], dv - lo, mask=m)
            eix = (j * 16 + lane) * EDGE_DIM
            for d in range(EDGE_DIM):
                ev = plsc.load_gather(eac, [eix + d])
                plsc.store_compressed(eabufs[d].at[pl.ds(p, 16)], ev, mask=m)
            return p + jnp.max(plsc.all_reduce_population_count(m))

        return lax.fori_loop(0, CH // 16, group, ptr)

    ptr = lax.fori_loop(0, E // CH, chunk, jnp.int32(0))

    zero16 = jnp.zeros((16,), jnp.int32)
    sent16 = jnp.full((16,), NPT, jnp.int32)
    zf16 = jnp.zeros((16,), jnp.float32)
    for gpad in range(KB // 16):
        sbuf[pl.ds(ptr + gpad * 16, 16)] = zero16
        dbuf[pl.ds(ptr + gpad * 16, 16)] = sent16
        for d in range(EDGE_DIM):
            eabufs[d][pl.ds(ptr + gpad * 16, 16)] = zf16

    cntv[...] = jnp.broadcast_to(ptr, (16,)).astype(jnp.int32)
    pltpu.sync_copy(sbuf.at[pl.ds(0, CAP)], sl_h.at[pl.ds(wid * CAP, CAP)])
    pltpu.sync_copy(dbuf.at[pl.ds(0, CAP)], dl_h.at[pl.ds(wid * CAP, CAP)])
    for d in range(EDGE_DIM):
        pltpu.sync_copy(eabufs[d].at[pl.ds(0, CAP)], eal_hs[d].at[pl.ds(wid * CAP, CAP)])
    pltpu.sync_copy(cntv, cnt_h.at[wid])


@_sc_lazy(
    out_type=jax.ShapeDtypeStruct((N_PAD, HP), jnp.float32),
    scratch_types=[
        pltpu.VMEM((N,), jnp.float32),
        pltpu.VMEM((NPT + 16,), jnp.float32),
        pltpu.VMEM((NPT + 16,), jnp.float32),
        pltpu.VMEM((NPT + 16,), jnp.float32),
        pltpu.VMEM((NPT + 16,), jnp.float32),
        pltpu.VMEM((NPT + 1, HP), jnp.float32),
        pltpu.VMEM((KB,), jnp.int32),
        pltpu.VMEM((KB,), jnp.int32),
        pltpu.VMEM((KB,), jnp.float32),
        pltpu.VMEM((KB, HP), jnp.float32),
        pltpu.VMEM((16,), jnp.float32),
        pltpu.VMEM((16,), jnp.int32),
        pltpu.SemaphoreType.DMA,
    ],
)
def _k_edge_gat(sl_h, dl_h, cnt_h, h_h, as_h, adp_h, asm_h, msg_h,
                astab, adl, mbl, sl, invl, acc, sbb, dbb, ebb, rows, asmv, cv, sem):
    wid = _wid()
    lo = wid * NPT
    zf = jnp.zeros((16,), jnp.float32)
    lane = lax.iota(jnp.int32, 16)

    pltpu.sync_copy(as_h, astab)
    pltpu.sync_copy(adp_h.at[pl.ds(lo, NPT)], adl.at[pl.ds(0, NPT)])
    pltpu.sync_copy(asm_h, asmv)
    pltpu.sync_copy(cnt_h.at[wid], cv)

    for j in range(NPT // 16):
        adv = adl[pl.ds(j * 16, 16)]
        mbl[pl.ds(j * 16, 16)] = _leaky(asmv[...] + adv)
        sl[pl.ds(j * 16, 16)] = zf

    def zrow(r, _):
        for j in range(HP // 16):
            acc[r, pl.ds(j * 16, 16)] = zf
        return 0
    lax.fori_loop(0, NPT + 1, zrow, 0)

    cnt = cv[pl.ds(0, 16)][0]
    nbat = (cnt + (KB - 1)) // KB

    def batch(b, _):
        off = wid * CAP + b * KB
        pltpu.sync_copy(sl_h.at[pl.ds(off, KB)], sbb)
        dma = pltpu.async_copy(h_h.at[sbb], rows, sem)
        pltpu.sync_copy(dl_h.at[pl.ds(off, KB)], dbb)

        def alpha_g(g, _2):
            s16 = pl.ds(g * 16, 16)
            sv = sbb[s16]
            dv = dbb[s16]
            asv = plsc.load_gather(astab, [sv])
            adv = plsc.load_gather(adl, [dv])
            mbv = plsc.load_gather(mbl, [dv])
            ev = jnp.exp(_leaky(asv + adv) - mbv)
            ev = jnp.where(dv >= NPT, 0.0, ev)
            ebb[s16] = ev
            plsc.addupdate_scatter(sl, [dv], ev)
            return 0
        lax.fori_loop(0, KB // 16, alpha_g, 0)
        dma.wait()

        def accum_g(g, _2):
            s16 = pl.ds(g * 16, 16)
            ev16 = ebb[s16]
            dv16 = dbb[s16]
            for l in range(16):
                e = ev16[l]
                d = dv16[l]
                for j in range(13):
                    sj = pl.ds(j * 16, 16)
                    plsc.addupdate(acc.at[d, sj], rows[g * 16 + l, sj] * e)
            return 0
        lax.fori_loop(0, KB // 16, accum_g, 0)
        return 0

    lax.fori_loop(0, nbat, batch, 0)

    for j in range(NPT // 16):
        sv = sl[pl.ds(j * 16, 16)]
        invl[pl.ds(j * 16, 16)] = jnp.where(sv > 0, 1.0 / sv, 0.0)

    def finrow(r, _):
        inv = invl[pl.ds(r, 16)][0]
        for j in range(13):
            acc[r, pl.ds(j * 16, 16)] *= inv
        return 0
    lax.fori_loop(0, NPT, finrow, 0)
    pltpu.sync_copy(acc.at[pl.ds(0, NPT)], msg_h.at[pl.ds(lo, NPT)])


@_sc_lazy(
    out_type=jax.ShapeDtypeStruct((N_PAD, HP), jnp.float32),
    scratch_types=[
        pltpu.VMEM((NPT + 16,), jnp.float32),
        pltpu.VMEM((NPT + 16,), jnp.float32),
        pltpu.VMEM((NPT + 16,), jnp.float32),
        pltpu.VMEM((NPT + 16,), jnp.float32),
        pltpu.VMEM((NPT + 1, HP), jnp.float32),
        pltpu.VMEM((KB,), jnp.int32),
        pltpu.VMEM((KB,), jnp.int32),
        pltpu.VMEM((KB,), jnp.float32), pltpu.VMEM((KB,), jnp.float32),
        pltpu.VMEM((KB,), jnp.float32), pltpu.VMEM((KB,), jnp.float32),
        pltpu.VMEM((KB,), jnp.float32),
        pltpu.VMEM((KB, HP), jnp.float32),
        pltpu.VMEM((KB, HP), jnp.float32),
        pltpu.VMEM((HP + 16,), jnp.float32),
        pltpu.VMEM((EDGE_DIM * HP + 16,), jnp.float32),
        pltpu.VMEM((16,), jnp.float32),
        pltpu.VMEM((16,), jnp.int32),
        pltpu.SemaphoreType.DMA,
        pltpu.SemaphoreType.DMA,
    ],
)
def _k_edge_gate(sl_h, dl_h, ea0_h, ea1_h, ea2_h, ea3_h, cnt_h, u_h, w_h,
                 attl_h, w1e_h, ar_h, bd_h, msg_h,
                 arl, mbl, sl, invl, acc, sbb, dbb, ea0b, ea1b, ea2b, ea3b,
                 ebb, urows, wrows, attl, w1ec, bdv, cv, sem1, sem2):
    ea_hs = (ea0_h, ea1_h, ea2_h, ea3_h)
    eabufs = (ea0b, ea1b, ea2b, ea3b)
    wid = _wid()
    lo = wid * NPT
    zf = jnp.zeros((16,), jnp.float32)
    lane = lax.iota(jnp.int32, 16)

    pltpu.sync_copy(ar_h.at[pl.ds(lo, NPT)], arl.at[pl.ds(0, NPT)])
    pltpu.sync_copy(attl_h, attl.at[pl.ds(0, HP)])
    pltpu.sync_copy(w1e_h, w1ec.at[pl.ds(0, EDGE_DIM * HP)])
    pltpu.sync_copy(bd_h, bdv)
    pltpu.sync_copy(cnt_h.at[wid], cv)

    for j in range(NPT // 16):
        arv = arl[pl.ds(j * 16, 16)]
        mbl[pl.ds(j * 16, 16)] = _leaky(bdv[...] + arv)
        sl[pl.ds(j * 16, 16)] = zf

    def zrow(r, _):
        for j in range(HP // 16):
            acc[r, pl.ds(j * 16, 16)] = zf
        return 0
    lax.fori_loop(0, NPT + 1, zrow, 0)

    cnt = cv[pl.ds(0, 16)][0]
    nbat = (cnt + (KB - 1)) // KB

    def batch(b, _):
        off = wid * CAP + b * KB
        pltpu.sync_copy(sl_h.at[pl.ds(off, KB)], sbb)
        dma_u = pltpu.async_copy(u_h.at[sbb], urows, sem1)
        dma_w = pltpu.async_copy(w_h.at[sbb], wrows, sem2)
        pltpu.sync_copy(dl_h.at[pl.ds(off, KB)], dbb)
        for d in range(EDGE_DIM):
            pltpu.sync_copy(ea_hs[d].at[pl.ds(off, KB)], eabufs[d])
        dma_u.wait()

        eav = [[eabufs[d][pl.ds(g * 16, 16)] for d in range(EDGE_DIM)]
               for g in range(KB // 16)]
        rowvs = [lane + g * 16 for g in range(KB // 16)]

        @plsc.parallel_loop(0, H, unroll=4, carry=(zf, zf, zf, zf))
        def dots(h, carry):
            hv = jnp.broadcast_to(h, (16,)).astype(jnp.int32)
            al_h = attl[pl.ds(h, 16)][0]
            ws = [w1ec[pl.ds(d * HP + h, 16)][0] for d in range(EDGE_DIM)]
            out = []
            for g in range(KB // 16):
                uv = plsc.load_gather(urows, [rowvs[g], hv])
                vv = eav[g][0] * ws[0]
                for d in range(1, EDGE_DIM):
                    vv += eav[g][d] * ws[d]
                out.append(carry[g] + al_h * _leaky(uv + vv))
            return tuple(out)

        for g in range(KB // 16):
            s16 = pl.ds(g * 16, 16)
            dv = dbb[s16]
            arv = plsc.load_gather(arl, [dv])
            mbv = plsc.load_gather(mbl, [dv])
            ev = jnp.exp(_leaky(dots[g] + arv) - mbv)
            ev = jnp.where(dv >= NPT, 0.0, ev)
            ebb[s16] = ev
            plsc.addupdate_scatter(sl, [dv], ev)
        dma_w.wait()

        def accum_g(g, _2):
            s16 = pl.ds(g * 16, 16)
            ev16 = ebb[s16]
            dv16 = dbb[s16]
            for l in range(16):
                e = ev16[l]
                d = dv16[l]
                for j in range(13):
                    sj = pl.ds(j * 16, 16)
                    plsc.addupdate(acc.at[d, sj], wrows[g * 16 + l, sj] * e)
            return 0
        lax.fori_loop(0, KB // 16, accum_g, 0)
        return 0

    lax.fori_loop(0, nbat, batch, 0)

    for j in range(NPT // 16):
        sv = sl[pl.ds(j * 16, 16)]
        invl[pl.ds(j * 16, 16)] = jnp.where(sv > 0, 1.0 / sv, 0.0)

    def finrow(r, _):
        inv = invl[pl.ds(r, 16)][0]
        for j in range(13):
            acc[r, pl.ds(j * 16, 16)] *= inv
        return 0
    lax.fori_loop(0, NPT, finrow, 0)
    pltpu.sync_copy(acc.at[pl.ds(0, NPT)], msg_h.at[pl.ds(lo, NPT)])


def _molpre_body(xv_ref, lw_ref, asrc_ref, batch_ref, hs_ref, as_ref, asmax_ref, out0_ref):
    i = pl.program_id(0)
    hs = jnp.dot(xv_ref[...], lw_ref[...].T, preferred_element_type=jnp.float32)
    hs_ref[...] = hs
    a_s = jnp.sum(hs * asrc_ref[...][None, :], axis=1)
    as_ref[...] = a_s[:, None]
    oneh = (batch_ref[...]
            == jax.lax.broadcasted_iota(jnp.int32, (1000, G), 1)).astype(jnp.float32)

    @pl.when(i == 0)
    def _():
        asmax_ref[...] = jnp.full((8, 128), -1e30, jnp.float32)
        out0_ref[...] = jnp.zeros((G, H), jnp.float32)
    asmax_ref[...] = jnp.maximum(asmax_ref[...], jnp.max(a_s))
    out0_ref[...] += jax.lax.dot_general(oneh, xv_ref[...], (((0,), (0,)), ((), ())),
                                         preferred_element_type=jnp.float32)

    @pl.when(i == pl.num_programs(0) - 1)
    def _():
        out0_ref[...] = jax.nn.relu(out0_ref[...])


def _molpre(xv, lw, asrc, batch):
    blk = 1000
    return pl.pallas_call(
        _molpre_body,
        grid=(N // blk,),
        in_specs=[
            pl.BlockSpec((blk, H), lambda i: (i, 0)),
            pl.BlockSpec((H, H), lambda i: (0, 0)),
            pl.BlockSpec((H,), lambda i: (0,)),
            pl.BlockSpec((blk, 1), lambda i: (i, 0)),
        ],
        out_specs=[
            pl.BlockSpec((blk, H), lambda i: (i, 0)),
            pl.BlockSpec((blk, 1), lambda i: (i, 0)),
            pl.BlockSpec((8, 128), lambda i: (0, 0)),
            pl.BlockSpec((G, H), lambda i: (0, 0)),
        ],
        out_shape=[
            jax.ShapeDtypeStruct((N, H), jnp.float32),
            jax.ShapeDtypeStruct((N, 1), jnp.float32),
            jax.ShapeDtypeStruct((8, 128), jnp.float32),
            jax.ShapeDtypeStruct((G, H), jnp.float32),
        ],
    )(xv, lw, asrc, batch)


def _mola_body(out_ref, lw_ref, adst_ref, asmax_ref, mb_ref, ad_ref):
    hd = jnp.dot(out_ref[...], lw_ref[...].T, preferred_element_type=jnp.float32)
    a_d = jnp.sum(hd * adst_ref[...][None, :], axis=1)
    ad_ref[...] = a_d
    mb_ref[...] = _leaky(jnp.max(asmax_ref[...]) + a_d)


def _mola(out, lw, adst, asmax):
    return pl.pallas_call(
        _mola_body,
        grid=(1,),
        in_specs=[
            pl.BlockSpec((G, H), lambda i: (0, 0)),
            pl.BlockSpec((H, H), lambda i: (0, 0)),
            pl.BlockSpec((H,), lambda i: (0,)),
            pl.BlockSpec((8, 128), lambda i: (0, 0)),
        ],
        out_specs=[pl.BlockSpec((G,), lambda i: (0,)), pl.BlockSpec((G,), lambda i: (0,))],
        out_shape=[jax.ShapeDtypeStruct((G,), jnp.float32),
                   jax.ShapeDtypeStruct((G,), jnp.float32)],
    )(out, lw, adst, asmax)


def _molmain_body(hs_ref, as_ref, batch_ref, ad_ref, mb_ref, s_ref, u_ref):
    i = pl.program_id(0)
    oneh = (batch_ref[...]
            == jax.lax.broadcasted_iota(jnp.int32, (1000, G), 1)).astype(jnp.float32)
    adn = oneh @ ad_ref[...]
    mbn = oneh @ mb_ref[...]
    e = jnp.exp(_leaky(as_ref[...][:, 0] + adn) - mbn)

    @pl.when(i == 0)
    def _():
        s_ref[...] = jnp.zeros((G,), jnp.float32)
        u_ref[...] = jnp.zeros((G, H), jnp.float32)
    s_ref[...] += e @ oneh
    u_ref[...] += jax.lax.dot_general(oneh, hs_ref[...] * e[:, None],
                                      (((0,), (0,)), ((), ())),
                                      preferred_element_type=jnp.float32)


def _molmain(hs, a_s, batch, a_d, mb):
    blk = 1000
    return pl.pallas_call(
        _molmain_body,
        grid=(N // blk,),
        in_specs=[
            pl.BlockSpec((blk, H), lambda i: (i, 0)),
            pl.BlockSpec((blk, 1), lambda i: (i, 0)),
            pl.BlockSpec((blk, 1), lambda i: (i, 0)),
            pl.BlockSpec((G,), lambda i: (0,)),
            pl.BlockSpec((G,), lambda i: (0,)),
        ],
        out_specs=[pl.BlockSpec((G,), lambda i: (0,)),
                   pl.BlockSpec((G, H), lambda i: (0, 0))],
        out_shape=[jax.ShapeDtypeStruct((G,), jnp.float32),
                   jax.ShapeDtypeStruct((G, H), jnp.float32)],
    )(hs, a_s, batch, a_d, mb)


def _molfin_body(u_ref, s_ref, out_ref, bias_ref, wr_ref, wz_ref, wn_ref,
                 vr_ref, vz_ref, vn_ref, br_ref, bz_ref, bn_ref,
                 cr_ref, cz_ref, cn_ref, l2w_ref, l2b_ref, outn_ref, pred_ref):
    s = s_ref[...]
    sinv = jnp.where(s > 0, 1.0 / s, 0.0)
    msg = u_ref[...] * sinv[:, None] + bias_ref[...][None, :]
    h = _elu(msg)
    xv = out_ref[...]
    dot = lambda a, b: jnp.dot(a, b.T, preferred_element_type=jnp.float32)
    i_r = dot(h, wr_ref[...]) + br_ref[...]
    i_z = dot(h, wz_ref[...]) + bz_ref[...]
    i_n = dot(h, wn_ref[...]) + bn_ref[...]
    h_r = dot(xv, vr_ref[...]) + cr_ref[...]
    h_z = dot(xv, vz_ref[...]) + cz_ref[...]
    h_n = dot(xv, vn_ref[...]) + cn_ref[...]
    r = jax.nn.sigmoid(i_r + h_r)
    z = jax.nn.sigmoid(i_z + h_z)
    n = jnp.tanh(i_n + r * h_n)
    outn = jax.nn.relu((1.0 - z) * n + z * xv)
    outn_ref[...] = outn
    pred_ref[...] = jnp.dot(outn, l2w_ref[...].T, preferred_element_type=jnp.float32) \
        + l2b_ref[...][None, :]


def _molfin(u, s, out, bias, gru_p, l2wp, l2bp):
    wih, whh, bih, bhh = gru_p["wih"], gru_p["whh"], gru_p["bih"], gru_p["bhh"]
    args = [u, s, out, bias,
            wih[:H], wih[H:2 * H], wih[2 * H:],
            whh[:H], whh[H:2 * H], whh[2 * H:],
            bih[:H], bih[H:2 * H], bih[2 * H:],
            bhh[:H], bhh[H:2 * H], bhh[2 * H:],
            l2wp, l2bp]
    mat = pl.BlockSpec((H, H), lambda i: (0, 0))
    vec = pl.BlockSpec((H,), lambda i: (0,))
    gh = pl.BlockSpec((G, H), lambda i: (0, 0))
    return pl.pallas_call(
        _molfin_body,
        grid=(1,),
        in_specs=[gh, pl.BlockSpec((G,), lambda i: (0,)), gh, vec,
                  mat, mat, mat, mat, mat, mat, vec, vec, vec, vec, vec, vec,
                  pl.BlockSpec((8, H), lambda i: (0, 0)),
                  pl.BlockSpec((8,), lambda i: (0,))],
        out_specs=[gh, pl.BlockSpec((G, 8), lambda i: (0, 0))],
        out_shape=[jax.ShapeDtypeStruct((G, H), jnp.float32),
                   jax.ShapeDtypeStruct((G, 8), jnp.float32)],
    )(*args)


def _pad_rows(w, rows):
    return jnp.pad(w, ((0, rows - w.shape[0]), (0, 0)))


def kernel(x, edge_index, edge_attr, batch, params):
    src, dst = edge_index[0], edge_index[1]
    g = params["gate"]
    w1 = g["lin1_w"]

    slist, dlist, ea0, ea1, ea2, ea3, cnt = _k_lists(src, dst, edge_attr.reshape(-1))

    w1xp = _pad_rows(w1[:, :H], HP)
    w2p = _pad_rows(g["lin2_w"], HP)
    xv, u, w, a_r, bmax = _embed(x, params["lin1_w"], params["lin1_b"],
                                 w1xp, w2p, g["att_r"], g["att_l"], w1[:, H:])
    arp = jnp.pad(a_r.reshape(-1), (0, N_PAD - N))
    bd16 = jnp.broadcast_to(bmax[0, 0], (16,))
    attlp = jnp.pad(g["att_l"], (0, HP - H))
    w1ecp = jnp.pad(w1[:, H:].T, ((0, 0), (0, HP - H)))

    msg = _k_edge_gate(slist, dlist, ea0, ea1, ea2, ea3, cnt, u, w,
                       attlp, w1ecp.reshape(-1), arp, bd16)
    xv = _gru(msg, g["bias"], xv, params["gru0"])

    for conv_p, gru_p in zip(params["atom_conv"], params["atom_gru"]):
        lwp = _pad_rows(conv_p["lin_w"], HP)
        h, a_s, a_d, asmax = _proj(xv, lwp, conv_p["att_src"], conv_p["att_dst"])
        adp = jnp.pad(a_d.reshape(-1), (0, N_PAD - N))
        asm16 = jnp.broadcast_to(asmax[0, 0], (16,))
        msg = _k_edge_gat(slist, dlist, cnt, h, a_s.reshape(-1), adp, asm16)
        xv = _gru(msg, conv_p["bias"], xv, gru_p)

    mp = params["mol_conv"]
    batch2 = batch[:, None]
    hs, a_s, asmax, out = _molpre(xv, mp["lin_w"], mp["att_src"], batch2)
    l2wp = _pad_rows(params["lin2_w"], 8)
    l2bp = jnp.pad(params["lin2_b"], (0, 7))
    pred = None
    for _ in range(3):
        mb, a_d = _mola(out, mp["lin_w"], mp["att_dst"], asmax)
        s, uacc = _molmain(hs, a_s, batch2, a_d, mb)
        out, pred8 = _molfin(uacc, s, out, mp["bias"], params["mol_gru"], l2wp, l2bp)
        pred = pred8[:, :1]
    return pred
